# Initial kernel scaffold; baseline (speedup 1.0000x reference)
import jax
import jax.numpy as jnp
from jax.experimental import pallas as pl


def kernel(x, edge_index, edge_label_index, snap, past1, past2, W1, Wr1, b1, W2, Wr2, b2, g1_Wi, g1_Wh, g1_bi, g1_bh, g2_Wi, g2_Wh, g2_bi, g2_bh, a1_W, a1_b, a1_q, a2_W, a2_b, a2_q, post_W, post_b, rel_emb):
    raise NotImplementedError("write your pallas kernel here")


# speedup vs baseline: 6.3629x; 6.3629x over previous
import functools

import jax
import jax.numpy as jnp
from jax import lax
from jax.experimental import pallas as pl
from jax.experimental.pallas import tpu as pltpu
from jax.experimental.pallas import tpu_sc as plsc

_N = 10000
_D = 128
_H1 = 128
_H2 = 64
_R = 3
_E = 320000
_L = 8192

_NC = 2
_NS = 16
_NW = _NC * _NS
_EPW = _E // _NW
_CW = 125
_CH = _EPW // _CW
_IB = 16
_NBATCH = _CH // _IB
_NPAD = 10240
_RPS = _NPAD // _NS
_NB = 1000
_GRID = _N // _NB


def _seg_sum_builder(width):
  mesh = plsc.VectorSubcoreMesh(core_axis_name="c", subcore_axis_name="s")
  out_type = [jax.ShapeDtypeStruct((_NC, _R, _NPAD, width), jnp.float32)]
  scratch = [
      pltpu.VMEM_SHARED((_NPAD, width), jnp.float32),
      pltpu.VMEM((_IB, _CW), jnp.int32),
      pltpu.VMEM((_IB, _CW), jnp.int32),
      pltpu.VMEM((_CW, width), jnp.float32),
      pltpu.SemaphoreType.DMA,
  ]

  def body(y0, y1, y2, src, dst, zw, out, acc, sidx, didx, rows, sem):
    cid = lax.axis_index("c")
    sid = lax.axis_index("s")
    wid = cid * _NS + sid
    for r, yr in enumerate((y0, y1, y2)):
      pltpu.sync_copy(zw, acc.at[pl.ds(sid * _RPS, _RPS)])
      plsc.subcore_barrier()

      def batch(b, _, yr=yr, r=r):
        pltpu.sync_copy(src.at[r, wid, b], sidx)
        pltpu.sync_copy(dst.at[r, wid, b], didx)

        def chunk(c, _):
          pltpu.async_copy(yr.at[sidx.at[c]], rows, sem).wait()
          pltpu.sync_copy(rows, acc.at[didx.at[c]], add=True)
          return _

        return lax.fori_loop(0, _IB, chunk, _)

      lax.fori_loop(0, _NBATCH, batch, None)
      plsc.subcore_barrier()
      pltpu.sync_copy(acc.at[pl.ds(sid * _RPS, _RPS)],
                      out.at[cid, r, pl.ds(sid * _RPS, _RPS)])
      plsc.subcore_barrier()

  return functools.partial(
      pl.kernel, mesh=mesh, out_type=out_type, scratch_types=scratch,
      compiler_params=pltpu.CompilerParams(use_tc_tiling_on_sc=False))(body)


_seg128 = _seg_sum_builder(_H1)
_seg64 = _seg_sum_builder(_H2)


def _deg_kernel():
  mesh = plsc.VectorSubcoreMesh(core_axis_name="c", subcore_axis_name="s")

  @functools.partial(
      pl.kernel, mesh=mesh,
      out_type=jax.ShapeDtypeStruct((_NC, _R, _NPAD, 16), jnp.float32),
      scratch_types=[
          pltpu.VMEM_SHARED((_NPAD, 16), jnp.float32),
          pltpu.VMEM((_IB, _CW), jnp.int32),
          pltpu.VMEM((_CW, 16), jnp.float32),
      ],
      compiler_params=pltpu.CompilerParams(use_tc_tiling_on_sc=False))
  def body(dst, z16, ones, outd, accd, didx, ones_v):
    cid = lax.axis_index("c")
    sid = lax.axis_index("s")
    wid = cid * _NS + sid
    pltpu.sync_copy(ones, ones_v)
    for r in range(_R):
      pltpu.sync_copy(z16, accd.at[pl.ds(sid * _RPS, _RPS)])
      plsc.subcore_barrier()

      def batch(b, _, r=r):
        pltpu.sync_copy(dst.at[r, wid, b], didx)

        def chunk(c, _):
          pltpu.sync_copy(ones_v, accd.at[didx.at[c]], add=True)
          return _

        return lax.fori_loop(0, _IB, chunk, _)

      lax.fori_loop(0, _NBATCH, batch, None)
      plsc.subcore_barrier()
      pltpu.sync_copy(accd.at[pl.ds(sid * _RPS, _RPS)],
                      outd.at[cid, r, pl.ds(sid * _RPS, _RPS)])
      plsc.subcore_barrier()

  return body


_deg = _deg_kernel()


def _score_kernel():
  mesh = plsc.VectorSubcoreMesh(core_axis_name="c", subcore_axis_name="s")
  lpw = _L // _NW

  @functools.partial(
      pl.kernel, mesh=mesh,
      out_type=jax.ShapeDtypeStruct((_R * _L,), jnp.float32),
      scratch_types=[
          pltpu.VMEM((_N,), jnp.float32),
          pltpu.VMEM((_N,), jnp.float32),
          pltpu.VMEM((lpw,), jnp.int32),
          pltpu.VMEM((lpw,), jnp.int32),
          pltpu.VMEM((16,), jnp.float32),
          pltpu.VMEM((16,), jnp.float32),
          pltpu.VMEM((lpw,), jnp.float32),
      ],
      compiler_params=pltpu.CompilerParams(needs_layout_passes=False))
  def body(p0, p1, hidx, tidx, relr, reli, out,
           p0_v, p1_v, hi_v, ti_v, rr_v, ri_v, sc_v):
    cid = lax.axis_index("c")
    sid = lax.axis_index("s")
    wid = cid * _NS + sid
    pltpu.sync_copy(p0, p0_v)
    pltpu.sync_copy(p1, p1_v)
    for r in range(_R):
      pltpu.sync_copy(relr.at[pl.ds(r * 16, 16)], rr_v)
      pltpu.sync_copy(reli.at[pl.ds(r * 16, 16)], ri_v)
      pltpu.sync_copy(hidx.at[pl.ds((r * _NW + wid) * lpw, lpw)], hi_v)
      pltpu.sync_copy(tidx.at[pl.ds((r * _NW + wid) * lpw, lpw)], ti_v)
      rr = rr_v[...]
      ri = ri_v[...]
      for g in range(lpw // 16):
        hvec = hi_v[pl.ds(g * 16, 16)]
        tvec = ti_v[pl.ds(g * 16, 16)]
        h0 = plsc.load_gather(p0_v, [hvec])
        h1 = plsc.load_gather(p1_v, [hvec])
        t0 = plsc.load_gather(p0_v, [tvec])
        t1 = plsc.load_gather(p1_v, [tvec])
        s = rr * (h0 * t0 + h1 * t1) + ri * (h0 * t1 - h1 * t0)
        sc_v[pl.ds(g * 16, 16)] = s
      pltpu.sync_copy(sc_v, out.at[pl.ds((r * _NW + wid) * lpw, lpw)])

  return body


_score = _score_kernel()


def _full(shape):
  return pl.BlockSpec(shape, lambda i: (0,) * len(shape))


def _tc0_body(x_ref, w1_ref, wr1_ref, b1_ref, y1_ref, root1_ref):
  x = x_ref[...]
  for r in range(_R):
    y1_ref[r] = jnp.dot(x, w1_ref[r], preferred_element_type=jnp.float32)
    root1_ref[r] = (jnp.dot(x, wr1_ref[r], preferred_element_type=jnp.float32)
                    + b1_ref[r])


def _gru_block(out_h, past, wi_ref, wh_ref, bi_ref, bh_ref, h):
  gi = jnp.dot(out_h, wi_ref[...], preferred_element_type=jnp.float32) + bi_ref[...]
  gh = jnp.dot(past, wh_ref[...], preferred_element_type=jnp.float32) + bh_ref[...]
  ir, iz, inn = gi[:, :h], gi[:, h:2 * h], gi[:, 2 * h:]
  hr, hz, hn = gh[:, :h], gh[:, h:2 * h], gh[:, 2 * h:]
  rg = jax.nn.sigmoid(ir + hr)
  zg = jax.nn.sigmoid(iz + hz)
  ng = jnp.tanh(inn + rg * hn)
  return (1.0 - zg) * ng + zg * past


def _layer_body(h, snap_ref, aggp_ref, degp_ref, root_ref, past_ref,
                wi_ref, wh_ref, bi_ref, bh_ref, aw_ref, ab_ref, aq_ref,
                cur_ref, wp_ref):
  snap0 = snap_ref[0:1, 0:1]
  lanes = lax.broadcasted_iota(jnp.int32, (8, 128), 1)
  acc = jnp.zeros((8, 128), jnp.float32)
  for r in range(_R):
    agg = aggp_ref[0, r] + aggp_ref[1, r]
    deg = degp_ref[0, r, :, 0:1] + degp_ref[1, r, :, 0:1]
    agg = agg / jnp.maximum(deg, 1.0)
    out_h = jnp.maximum(agg + root_ref[r], 0.0)
    g = _gru_block(out_h, past_ref[r], wi_ref, wh_ref, bi_ref, bh_ref, h)
    cur = jnp.where(snap0 == 0.0, out_h, g)
    cur_ref[r] = cur
    w = jnp.dot(jnp.tanh(jnp.dot(cur, aw_ref[...],
                                 preferred_element_type=jnp.float32)
                         + ab_ref[...]),
                aq_ref[...], preferred_element_type=jnp.float32)
    s = jnp.sum(w[:, 0:1])
    acc = acc + jnp.where(lanes == r, s, 0.0)

  @pl.when(pl.program_id(0) == 0)
  def _init():
    wp_ref[...] = jnp.zeros((8, 128), jnp.float32)

  wp_ref[...] += acc


def _tc2_body(cur1_ref, beta_ref, w2_ref, wr2_ref, b2_ref, y2_ref, root2_ref):
  h1 = cur1_ref[0] * beta_ref[0:1, 0:1]
  h1 = h1 + cur1_ref[1] * beta_ref[1:2, 0:1]
  h1 = h1 + cur1_ref[2] * beta_ref[2:3, 0:1]
  for r in range(_R):
    y2_ref[r] = jnp.dot(h1, w2_ref[r], preferred_element_type=jnp.float32)
    root2_ref[r] = (jnp.dot(h1, wr2_ref[r], preferred_element_type=jnp.float32)
                    + b2_ref[r])


def _tc4_body(cur2_ref, beta_ref, pw_ref, pb_ref, out_ref):
  h2 = cur2_ref[0] * beta_ref[0:1, 0:1]
  h2 = h2 + cur2_ref[1] * beta_ref[1:2, 0:1]
  h2 = h2 + cur2_ref[2] * beta_ref[2:3, 0:1]
  out_ref[...] = (jnp.dot(h2, pw_ref[...], preferred_element_type=jnp.float32)
                  + pb_ref[...])


def kernel(x, edge_index, edge_label_index, snap, past1, past2, W1, Wr1, b1,
           W2, Wr2, b2, g1_Wi, g1_Wh, g1_bi, g1_bh, g2_Wi, g2_Wh, g2_bi,
           g2_bh, a1_W, a1_b, a1_q, a2_W, a2_b, a2_q, post_W, post_b,
           rel_emb):
  f32 = jnp.float32
  snapf = jnp.full((1, 128), snap, f32)
  src = edge_index[:, 0, :].reshape(_R, _NW, _NBATCH, _IB, _CW)
  dst = edge_index[:, 1, :].reshape(_R, _NW, _NBATCH, _IB, _CW)
  zw1 = jnp.zeros((_RPS, _H1), f32)
  zw2 = jnp.zeros((_RPS, _H2), f32)
  z16 = jnp.zeros((_RPS, 16), f32)
  ones = jnp.zeros((_CW, 16), f32).at[:, 0].set(1.0)

  degp = _deg(dst, z16, ones)

  nblk = pl.BlockSpec((_NB, _D), lambda i: (i, 0))

  y1, root1 = pl.pallas_call(
      _tc0_body,
      grid=(_GRID,),
      in_specs=[nblk, _full((_R, _D, _H1)), _full((_R, _D, _H1)),
                _full((_R, 1, _H1))],
      out_specs=[pl.BlockSpec((_R, _NB, _H1), lambda i: (0, i, 0))] * 2,
      out_shape=[jax.ShapeDtypeStruct((_R, _N, _H1), f32)] * 2,
  )(x, W1, Wr1, b1[:, None, :])

  (agg1p,) = _seg128(y1[0], y1[1], y1[2], src, dst, zw1)

  layer1 = functools.partial(_layer_body, _H1)
  cur1, wp1 = pl.pallas_call(
      layer1,
      grid=(_GRID,),
      in_specs=[
          _full((1, 128)),
          pl.BlockSpec((_NC, _R, _NB, _H1), lambda i: (0, 0, i, 0)),
          pl.BlockSpec((_NC, _R, _NB, 16), lambda i: (0, 0, i, 0)),
          pl.BlockSpec((_R, _NB, _H1), lambda i: (0, i, 0)),
          pl.BlockSpec((_R, _NB, _H1), lambda i: (0, i, 0)),
          _full((_H1, 3 * _H1)), _full((_H1, 3 * _H1)),
          _full((1, 3 * _H1)), _full((1, 3 * _H1)),
          _full((_H1, _H1)), _full((1, _H1)), _full((_H1, 8)),
      ],
      out_specs=[pl.BlockSpec((_R, _NB, _H1), lambda i: (0, i, 0)),
                 pl.BlockSpec((8, 128), lambda i: (0, 0))],
      out_shape=[jax.ShapeDtypeStruct((_R, _N, _H1), f32),
                 jax.ShapeDtypeStruct((8, 128), f32)],
  )(snapf, agg1p, degp, root1, past1, g1_Wi, g1_Wh, g1_bi[None, :],
    g1_bh[None, :], a1_W, a1_b[None, :],
    jnp.zeros((_H1, 8), f32).at[:, 0].set(a1_q))

  beta1 = jax.nn.softmax(wp1[0, :_R] / _N)
  beta1b = jnp.broadcast_to(beta1[:, None], (_R, 128))

  y2, root2 = pl.pallas_call(
      _tc2_body,
      grid=(_GRID,),
      in_specs=[pl.BlockSpec((_R, _NB, _H1), lambda i: (0, i, 0)),
                _full((_R, 128)), _full((_R, _H1, _H2)),
                _full((_R, _H1, _H2)), _full((_R, 1, _H2))],
      out_specs=[pl.BlockSpec((_R, _NB, _H2), lambda i: (0, i, 0))] * 2,
      out_shape=[jax.ShapeDtypeStruct((_R, _N, _H2), f32)] * 2,
  )(cur1, beta1b, W2, Wr2, b2[:, None, :])

  (agg2p,) = _seg64(y2[0], y2[1], y2[2], src, dst, zw2)

  layer2 = functools.partial(_layer_body, _H2)
  cur2, wp2 = pl.pallas_call(
      layer2,
      grid=(_GRID,),
      in_specs=[
          _full((1, 128)),
          pl.BlockSpec((_NC, _R, _NB, _H2), lambda i: (0, 0, i, 0)),
          pl.BlockSpec((_NC, _R, _NB, 16), lambda i: (0, 0, i, 0)),
          pl.BlockSpec((_R, _NB, _H2), lambda i: (0, i, 0)),
          pl.BlockSpec((_R, _NB, _H2), lambda i: (0, i, 0)),
          _full((_H2, 3 * _H2)), _full((_H2, 3 * _H2)),
          _full((1, 3 * _H2)), _full((1, 3 * _H2)),
          _full((_H2, _H2)), _full((1, _H2)), _full((_H2, 8)),
      ],
      out_specs=[pl.BlockSpec((_R, _NB, _H2), lambda i: (0, i, 0)),
                 pl.BlockSpec((8, 128), lambda i: (0, 0))],
      out_shape=[jax.ShapeDtypeStruct((_R, _N, _H2), f32),
                 jax.ShapeDtypeStruct((8, 128), f32)],
  )(snapf, agg2p, degp, root2, past2, g2_Wi, g2_Wh, g2_bi[None, :],
    g2_bh[None, :], a2_W, a2_b[None, :],
    jnp.zeros((_H2, 8), f32).at[:, 0].set(a2_q))

  beta2 = jax.nn.softmax(wp2[0, :_R] / _N)
  beta2b = jnp.broadcast_to(beta2[:, None], (_R, 128))

  pwp = jnp.zeros((_H2, 16), f32).at[:, :2].set(post_W)
  pbp = jnp.zeros((1, 16), f32).at[0, :2].set(post_b)
  outp = pl.pallas_call(
      _tc4_body,
      grid=(_GRID,),
      in_specs=[pl.BlockSpec((_R, _NB, _H2), lambda i: (0, i, 0)),
                _full((_R, 128)), _full((_H2, 16)), _full((1, 16))],
      out_specs=pl.BlockSpec((_NB, 16), lambda i: (i, 0)),
      out_shape=jax.ShapeDtypeStruct((_N, 16), f32),
  )(cur2, beta2b, pwp, pbp)

  hidx = edge_label_index[:, 0, :].reshape(-1)
  tidx = edge_label_index[:, 1, :].reshape(-1)
  relr = jnp.broadcast_to(rel_emb[:, 0:1], (_R, 16)).reshape(-1)
  reli = jnp.broadcast_to(rel_emb[:, 1:2], (_R, 16)).reshape(-1)
  scores = _score(outp[:, 0], outp[:, 1], hidx, tidx, relr, reli)

  return scores.reshape(_R, _L), cur1, cur2


# speedup vs baseline: 7.7073x; 1.2113x over previous
import functools

import jax
import jax.numpy as jnp
from jax import lax
from jax.experimental import pallas as pl
from jax.experimental.pallas import tpu as pltpu
from jax.experimental.pallas import tpu_sc as plsc

_N = 10000
_D = 128
_H1 = 128
_H2 = 64
_R = 3
_E = 320000
_L = 8192

_NC = 2
_NS = 16
_NW = _NC * _NS
_EPW = _E // _NW
_CW = 100
_CH = _EPW // _CW
_IB = 10
_NBATCH = _CH // _IB
_NPAD = 10240
_RPS = _NPAD // _NS
_NB = 1000
_GRID = _N // _NB


def _seg_sum_builder(width):
  mesh = plsc.VectorSubcoreMesh(core_axis_name="c", subcore_axis_name="s")
  out_type = [jax.ShapeDtypeStruct((_NC, _R, _NPAD, width), jnp.float32)]
  scratch = [
      pltpu.VMEM_SHARED((_NPAD, width), jnp.float32),
      pltpu.VMEM((_IB, _CW), jnp.int32),
      pltpu.VMEM((_IB, _CW), jnp.int32),
      pltpu.VMEM((_CW, width), jnp.float32),
      pltpu.VMEM((_CW, width), jnp.float32),
      pltpu.SemaphoreType.DMA,
      pltpu.SemaphoreType.DMA,
  ]

  def body(y0, y1, y2, src, dst, zw, out, acc,
           sidx, didx, rows_a, rows_b, sem_a, sem_b):
    cid = lax.axis_index("c")
    sid = lax.axis_index("s")
    wid = cid * _NS + sid
    for r, yr in enumerate((y0, y1, y2)):
      pltpu.sync_copy(zw, acc.at[pl.ds(sid * _RPS, _RPS)])
      plsc.subcore_barrier()

      def batch(b, _, yr=yr, r=r):
        pltpu.sync_copy(src.at[r, wid, b], sidx)
        pltpu.sync_copy(dst.at[r, wid, b], didx)
        pltpu.async_copy(yr.at[sidx.at[0]], rows_a, sem_a)

        def pair(p, _):
          c0 = 2 * p
          c1 = c0 + 1
          pltpu.async_copy(yr.at[sidx.at[c1]], rows_b, sem_b)
          pltpu.make_async_copy(yr.at[sidx.at[c0]], rows_a, sem_a).wait()
          pltpu.sync_copy(rows_a, acc.at[didx.at[c0]], add=True)

          @pl.when(c1 + 1 < _IB)
          def _prefetch():
            pltpu.async_copy(yr.at[sidx.at[c1 + 1]], rows_a, sem_a)

          pltpu.make_async_copy(yr.at[sidx.at[c1]], rows_b, sem_b).wait()
          pltpu.sync_copy(rows_b, acc.at[didx.at[c1]], add=True)
          return _

        return lax.fori_loop(0, _IB // 2, pair, _)

      lax.fori_loop(0, _NBATCH, batch, None)
      plsc.subcore_barrier()
      pltpu.sync_copy(acc.at[pl.ds(sid * _RPS, _RPS)],
                      out.at[cid, r, pl.ds(sid * _RPS, _RPS)])
      plsc.subcore_barrier()

  return functools.partial(
      pl.kernel, mesh=mesh, out_type=out_type, scratch_types=scratch,
      compiler_params=pltpu.CompilerParams(use_tc_tiling_on_sc=False))(body)


_seg128 = _seg_sum_builder(_H1)
_seg64 = _seg_sum_builder(_H2)


def _deg_kernel():
  mesh = plsc.VectorSubcoreMesh(core_axis_name="c", subcore_axis_name="s")

  @functools.partial(
      pl.kernel, mesh=mesh,
      out_type=jax.ShapeDtypeStruct((_NC, _R, _NPAD, 16), jnp.float32),
      scratch_types=[
          pltpu.VMEM_SHARED((_NPAD, 16), jnp.float32),
          pltpu.VMEM((_IB, _CW), jnp.int32),
          pltpu.VMEM((_CW, 16), jnp.float32),
      ],
      compiler_params=pltpu.CompilerParams(use_tc_tiling_on_sc=False))
  def body(dst, z16, ones, outd, accd, didx, ones_v):
    cid = lax.axis_index("c")
    sid = lax.axis_index("s")
    wid = cid * _NS + sid
    pltpu.sync_copy(ones, ones_v)
    for r in range(_R):
      pltpu.sync_copy(z16, accd.at[pl.ds(sid * _RPS, _RPS)])
      plsc.subcore_barrier()

      def batch(b, _, r=r):
        pltpu.sync_copy(dst.at[r, wid, b], didx)

        def chunk(c, _):
          pltpu.sync_copy(ones_v, accd.at[didx.at[c]], add=True)
          return _

        return lax.fori_loop(0, _IB, chunk, _)

      lax.fori_loop(0, _NBATCH, batch, None)
      plsc.subcore_barrier()
      pltpu.sync_copy(accd.at[pl.ds(sid * _RPS, _RPS)],
                      outd.at[cid, r, pl.ds(sid * _RPS, _RPS)])
      plsc.subcore_barrier()

  return body


_deg = _deg_kernel()


def _score_kernel():
  mesh = plsc.VectorSubcoreMesh(core_axis_name="c", subcore_axis_name="s")
  lpw = _L // _NW

  @functools.partial(
      pl.kernel, mesh=mesh,
      out_type=jax.ShapeDtypeStruct((_R * _L,), jnp.float32),
      scratch_types=[
          pltpu.VMEM((_N,), jnp.float32),
          pltpu.VMEM((_N,), jnp.float32),
          pltpu.VMEM((lpw,), jnp.int32),
          pltpu.VMEM((lpw,), jnp.int32),
          pltpu.VMEM((16,), jnp.float32),
          pltpu.VMEM((16,), jnp.float32),
          pltpu.VMEM((lpw,), jnp.float32),
      ],
      compiler_params=pltpu.CompilerParams(needs_layout_passes=False))
  def body(p0, p1, hidx, tidx, relr, reli, out,
           p0_v, p1_v, hi_v, ti_v, rr_v, ri_v, sc_v):
    cid = lax.axis_index("c")
    sid = lax.axis_index("s")
    wid = cid * _NS + sid
    pltpu.sync_copy(p0, p0_v)
    pltpu.sync_copy(p1, p1_v)
    for r in range(_R):
      pltpu.sync_copy(relr.at[pl.ds(r * 16, 16)], rr_v)
      pltpu.sync_copy(reli.at[pl.ds(r * 16, 16)], ri_v)
      pltpu.sync_copy(hidx.at[pl.ds((r * _NW + wid) * lpw, lpw)], hi_v)
      pltpu.sync_copy(tidx.at[pl.ds((r * _NW + wid) * lpw, lpw)], ti_v)
      rr = rr_v[...]
      ri = ri_v[...]
      for g in range(lpw // 16):
        hvec = hi_v[pl.ds(g * 16, 16)]
        tvec = ti_v[pl.ds(g * 16, 16)]
        h0 = plsc.load_gather(p0_v, [hvec])
        h1 = plsc.load_gather(p1_v, [hvec])
        t0 = plsc.load_gather(p0_v, [tvec])
        t1 = plsc.load_gather(p1_v, [tvec])
        s = rr * (h0 * t0 + h1 * t1) + ri * (h0 * t1 - h1 * t0)
        sc_v[pl.ds(g * 16, 16)] = s
      pltpu.sync_copy(sc_v, out.at[pl.ds((r * _NW + wid) * lpw, lpw)])

  return body


_score = _score_kernel()


def _full(shape):
  return pl.BlockSpec(shape, lambda i: (0,) * len(shape))


def _tc0_body(x_ref, w1_ref, wr1_ref, b1_ref, y1_ref, root1_ref):
  x = x_ref[...]
  for r in range(_R):
    y1_ref[r] = jnp.dot(x, w1_ref[r], preferred_element_type=jnp.float32)
    root1_ref[r] = (jnp.dot(x, wr1_ref[r], preferred_element_type=jnp.float32)
                    + b1_ref[r])


def _gru_block(out_h, past, wi_ref, wh_ref, bi_ref, bh_ref, h):
  gi = jnp.dot(out_h, wi_ref[...], preferred_element_type=jnp.float32) + bi_ref[...]
  gh = jnp.dot(past, wh_ref[...], preferred_element_type=jnp.float32) + bh_ref[...]
  ir, iz, inn = gi[:, :h], gi[:, h:2 * h], gi[:, 2 * h:]
  hr, hz, hn = gh[:, :h], gh[:, h:2 * h], gh[:, 2 * h:]
  rg = jax.nn.sigmoid(ir + hr)
  zg = jax.nn.sigmoid(iz + hz)
  ng = jnp.tanh(inn + rg * hn)
  return (1.0 - zg) * ng + zg * past


def _layer_body(h, snap_ref, aggp_ref, degp_ref, root_ref, past_ref,
                wi_ref, wh_ref, bi_ref, bh_ref, aw_ref, ab_ref, aq_ref,
                cur_ref, wp_ref):
  snap0 = snap_ref[0:1, 0:1]
  lanes = lax.broadcasted_iota(jnp.int32, (8, 128), 1)
  acc = jnp.zeros((8, 128), jnp.float32)
  for r in range(_R):
    agg = aggp_ref[0, r] + aggp_ref[1, r]
    deg = degp_ref[0, r, :, 0:1] + degp_ref[1, r, :, 0:1]
    agg = agg / jnp.maximum(deg, 1.0)
    out_h = jnp.maximum(agg + root_ref[r], 0.0)
    g = _gru_block(out_h, past_ref[r], wi_ref, wh_ref, bi_ref, bh_ref, h)
    cur = jnp.where(snap0 == 0.0, out_h, g)
    cur_ref[r] = cur
    w = jnp.dot(jnp.tanh(jnp.dot(cur, aw_ref[...],
                                 preferred_element_type=jnp.float32)
                         + ab_ref[...]),
                aq_ref[...], preferred_element_type=jnp.float32)
    s = jnp.sum(w[:, 0:1])
    acc = acc + jnp.where(lanes == r, s, 0.0)

  @pl.when(pl.program_id(0) == 0)
  def _init():
    wp_ref[...] = jnp.zeros((8, 128), jnp.float32)

  wp_ref[...] += acc


def _tc2_body(cur1_ref, beta_ref, w2_ref, wr2_ref, b2_ref, y2_ref, root2_ref):
  h1 = cur1_ref[0] * beta_ref[0:1, 0:1]
  h1 = h1 + cur1_ref[1] * beta_ref[1:2, 0:1]
  h1 = h1 + cur1_ref[2] * beta_ref[2:3, 0:1]
  for r in range(_R):
    y2_ref[r] = jnp.dot(h1, w2_ref[r], preferred_element_type=jnp.float32)
    root2_ref[r] = (jnp.dot(h1, wr2_ref[r], preferred_element_type=jnp.float32)
                    + b2_ref[r])


def _tc4_body(cur2_ref, beta_ref, pw_ref, pb_ref, out_ref):
  h2 = cur2_ref[0] * beta_ref[0:1, 0:1]
  h2 = h2 + cur2_ref[1] * beta_ref[1:2, 0:1]
  h2 = h2 + cur2_ref[2] * beta_ref[2:3, 0:1]
  out_ref[...] = (jnp.dot(h2, pw_ref[...], preferred_element_type=jnp.float32)
                  + pb_ref[...])


def kernel(x, edge_index, edge_label_index, snap, past1, past2, W1, Wr1, b1,
           W2, Wr2, b2, g1_Wi, g1_Wh, g1_bi, g1_bh, g2_Wi, g2_Wh, g2_bi,
           g2_bh, a1_W, a1_b, a1_q, a2_W, a2_b, a2_q, post_W, post_b,
           rel_emb):
  f32 = jnp.float32
  snapf = jnp.full((1, 128), snap, f32)
  src = edge_index[:, 0, :].reshape(_R, _NW, _NBATCH, _IB, _CW)
  dst = edge_index[:, 1, :].reshape(_R, _NW, _NBATCH, _IB, _CW)
  zw1 = jnp.zeros((_RPS, _H1), f32)
  zw2 = jnp.zeros((_RPS, _H2), f32)
  z16 = jnp.zeros((_RPS, 16), f32)
  ones = jnp.zeros((_CW, 16), f32).at[:, 0].set(1.0)

  degp = _deg(dst, z16, ones)

  nblk = pl.BlockSpec((_NB, _D), lambda i: (i, 0))

  y1, root1 = pl.pallas_call(
      _tc0_body,
      grid=(_GRID,),
      in_specs=[nblk, _full((_R, _D, _H1)), _full((_R, _D, _H1)),
                _full((_R, 1, _H1))],
      out_specs=[pl.BlockSpec((_R, _NB, _H1), lambda i: (0, i, 0))] * 2,
      out_shape=[jax.ShapeDtypeStruct((_R, _N, _H1), f32)] * 2,
  )(x, W1, Wr1, b1[:, None, :])

  (agg1p,) = _seg128(y1[0], y1[1], y1[2], src, dst, zw1)

  layer1 = functools.partial(_layer_body, _H1)
  cur1, wp1 = pl.pallas_call(
      layer1,
      grid=(_GRID,),
      in_specs=[
          _full((1, 128)),
          pl.BlockSpec((_NC, _R, _NB, _H1), lambda i: (0, 0, i, 0)),
          pl.BlockSpec((_NC, _R, _NB, 16), lambda i: (0, 0, i, 0)),
          pl.BlockSpec((_R, _NB, _H1), lambda i: (0, i, 0)),
          pl.BlockSpec((_R, _NB, _H1), lambda i: (0, i, 0)),
          _full((_H1, 3 * _H1)), _full((_H1, 3 * _H1)),
          _full((1, 3 * _H1)), _full((1, 3 * _H1)),
          _full((_H1, _H1)), _full((1, _H1)), _full((_H1, 8)),
      ],
      out_specs=[pl.BlockSpec((_R, _NB, _H1), lambda i: (0, i, 0)),
                 pl.BlockSpec((8, 128), lambda i: (0, 0))],
      out_shape=[jax.ShapeDtypeStruct((_R, _N, _H1), f32),
                 jax.ShapeDtypeStruct((8, 128), f32)],
  )(snapf, agg1p, degp, root1, past1, g1_Wi, g1_Wh, g1_bi[None, :],
    g1_bh[None, :], a1_W, a1_b[None, :],
    jnp.zeros((_H1, 8), f32).at[:, 0].set(a1_q))

  beta1 = jax.nn.softmax(wp1[0, :_R] / _N)
  beta1b = jnp.broadcast_to(beta1[:, None], (_R, 128))

  y2, root2 = pl.pallas_call(
      _tc2_body,
      grid=(_GRID,),
      in_specs=[pl.BlockSpec((_R, _NB, _H1), lambda i: (0, i, 0)),
                _full((_R, 128)), _full((_R, _H1, _H2)),
                _full((_R, _H1, _H2)), _full((_R, 1, _H2))],
      out_specs=[pl.BlockSpec((_R, _NB, _H2), lambda i: (0, i, 0))] * 2,
      out_shape=[jax.ShapeDtypeStruct((_R, _N, _H2), f32)] * 2,
  )(cur1, beta1b, W2, Wr2, b2[:, None, :])

  (agg2p,) = _seg64(y2[0], y2[1], y2[2], src, dst, zw2)

  layer2 = functools.partial(_layer_body, _H2)
  cur2, wp2 = pl.pallas_call(
      layer2,
      grid=(_GRID,),
      in_specs=[
          _full((1, 128)),
          pl.BlockSpec((_NC, _R, _NB, _H2), lambda i: (0, 0, i, 0)),
          pl.BlockSpec((_NC, _R, _NB, 16), lambda i: (0, 0, i, 0)),
          pl.BlockSpec((_R, _NB, _H2), lambda i: (0, i, 0)),
          pl.BlockSpec((_R, _NB, _H2), lambda i: (0, i, 0)),
          _full((_H2, 3 * _H2)), _full((_H2, 3 * _H2)),
          _full((1, 3 * _H2)), _full((1, 3 * _H2)),
          _full((_H2, _H2)), _full((1, _H2)), _full((_H2, 8)),
      ],
      out_specs=[pl.BlockSpec((_R, _NB, _H2), lambda i: (0, i, 0)),
                 pl.BlockSpec((8, 128), lambda i: (0, 0))],
      out_shape=[jax.ShapeDtypeStruct((_R, _N, _H2), f32),
                 jax.ShapeDtypeStruct((8, 128), f32)],
  )(snapf, agg2p, degp, root2, past2, g2_Wi, g2_Wh, g2_bi[None, :],
    g2_bh[None, :], a2_W, a2_b[None, :],
    jnp.zeros((_H2, 8), f32).at[:, 0].set(a2_q))

  beta2 = jax.nn.softmax(wp2[0, :_R] / _N)
  beta2b = jnp.broadcast_to(beta2[:, None], (_R, 128))

  pwp = jnp.zeros((_H2, 16), f32).at[:, :2].set(post_W)
  pbp = jnp.zeros((1, 16), f32).at[0, :2].set(post_b)
  outp = pl.pallas_call(
      _tc4_body,
      grid=(_GRID,),
      in_specs=[pl.BlockSpec((_R, _NB, _H2), lambda i: (0, i, 0)),
                _full((_R, 128)), _full((_H2, 16)), _full((1, 16))],
      out_specs=pl.BlockSpec((_NB, 16), lambda i: (i, 0)),
      out_shape=jax.ShapeDtypeStruct((_N, 16), f32),
  )(cur2, beta2b, pwp, pbp)

  hidx = edge_label_index[:, 0, :].reshape(-1)
  tidx = edge_label_index[:, 1, :].reshape(-1)
  relr = jnp.broadcast_to(rel_emb[:, 0:1], (_R, 16)).reshape(-1)
  reli = jnp.broadcast_to(rel_emb[:, 1:2], (_R, 16)).reshape(-1)
  scores = _score(outp[:, 0], outp[:, 1], hidx, tidx, relr, reli)

  return scores.reshape(_R, _L), cur1, cur2


# speedup vs baseline: 7.8120x; 1.0136x over previous
import functools

import jax
import jax.numpy as jnp
from jax import lax
from jax.experimental import pallas as pl
from jax.experimental.pallas import tpu as pltpu
from jax.experimental.pallas import tpu_sc as plsc

_N = 10000
_D = 128
_H1 = 128
_H2 = 64
_R = 3
_E = 320000
_L = 8192

_NC = 2
_NS = 16
_NW = _NC * _NS
_EPW = _E // _NW
_CW = 100
_CH = _EPW // _CW
_IB = 10
_NBATCH = _CH // _IB
_NPAD = 10240
_RPS = _NPAD // _NS
_NB = 1000
_GRID = _N // _NB


def _seg_sum_builder(width):
  mesh = plsc.VectorSubcoreMesh(core_axis_name="c", subcore_axis_name="s")
  out_type = [jax.ShapeDtypeStruct((_NC, _R, _NPAD, width), jnp.float32)]
  scratch = [
      pltpu.VMEM_SHARED((_NPAD, width), jnp.float32),
      pltpu.VMEM((_IB, _CW), jnp.int32),
      pltpu.VMEM((_IB, _CW), jnp.int32),
      pltpu.VMEM((_CW, width), jnp.float32),
      pltpu.VMEM((_CW, width), jnp.float32),
      pltpu.SemaphoreType.DMA,
      pltpu.SemaphoreType.DMA,
  ]

  def body(y0, y1, y2, src, dst, zw, out, acc,
           sidx, didx, rows_a, rows_b, sem_a, sem_b):
    cid = lax.axis_index("c")
    sid = lax.axis_index("s")
    wid = cid * _NS + sid
    for r, yr in enumerate((y0, y1, y2)):
      pltpu.sync_copy(zw, acc.at[pl.ds(sid * _RPS, _RPS)])
      plsc.subcore_barrier()

      def batch(b, _, yr=yr, r=r):
        pltpu.sync_copy(src.at[r, wid, b], sidx)
        pltpu.sync_copy(dst.at[r, wid, b], didx)
        pltpu.async_copy(yr.at[sidx.at[0]], rows_a, sem_a)

        def pair(p, _):
          c0 = 2 * p
          c1 = c0 + 1
          pltpu.async_copy(yr.at[sidx.at[c1]], rows_b, sem_b)
          pltpu.make_async_copy(yr.at[sidx.at[c0]], rows_a, sem_a).wait()
          pltpu.sync_copy(rows_a, acc.at[didx.at[c0]], add=True)

          @pl.when(c1 + 1 < _IB)
          def _prefetch():
            pltpu.async_copy(yr.at[sidx.at[c1 + 1]], rows_a, sem_a)

          pltpu.make_async_copy(yr.at[sidx.at[c1]], rows_b, sem_b).wait()
          pltpu.sync_copy(rows_b, acc.at[didx.at[c1]], add=True)
          return _

        return lax.fori_loop(0, _IB // 2, pair, _)

      lax.fori_loop(0, _NBATCH, batch, None)
      plsc.subcore_barrier()
      pltpu.sync_copy(acc.at[pl.ds(sid * _RPS, _RPS)],
                      out.at[cid, r, pl.ds(sid * _RPS, _RPS)])
      plsc.subcore_barrier()

  return functools.partial(
      pl.kernel, mesh=mesh, out_type=out_type, scratch_types=scratch,
      compiler_params=pltpu.CompilerParams(use_tc_tiling_on_sc=False))(body)


_seg128 = _seg_sum_builder(_H1)
_seg64 = _seg_sum_builder(_H2)


def _deg_kernel():
  mesh = plsc.VectorSubcoreMesh(core_axis_name="c", subcore_axis_name="s")

  @functools.partial(
      pl.kernel, mesh=mesh,
      out_type=jax.ShapeDtypeStruct((_NC, _R, _NPAD, 16), jnp.float32),
      scratch_types=[
          pltpu.VMEM_SHARED((_NPAD, 16), jnp.float32),
          pltpu.VMEM((_IB, _CW), jnp.int32),
          pltpu.VMEM((_CW, 16), jnp.float32),
          pltpu.SemaphoreType.DMA,
      ],
      compiler_params=pltpu.CompilerParams(use_tc_tiling_on_sc=False))
  def body(dst, z16, ones, outd, accd, didx, ones_v, sem):
    cid = lax.axis_index("c")
    sid = lax.axis_index("s")
    wid = cid * _NS + sid
    pltpu.sync_copy(ones, ones_v)
    for r in range(_R):
      pltpu.sync_copy(z16, accd.at[pl.ds(sid * _RPS, _RPS)])
      plsc.subcore_barrier()

      def batch(b, _, r=r):
        pltpu.sync_copy(dst.at[r, wid, b], didx)
        handles = [
            pltpu.async_copy(ones_v, accd.at[didx.at[c]], sem, add=True)
            for c in range(_IB)]
        for h in handles:
          h.wait()
        return _

      lax.fori_loop(0, _NBATCH, batch, None)
      plsc.subcore_barrier()
      pltpu.sync_copy(accd.at[pl.ds(sid * _RPS, _RPS)],
                      outd.at[cid, r, pl.ds(sid * _RPS, _RPS)])
      plsc.subcore_barrier()

  return body


_deg = _deg_kernel()


def _score_kernel():
  mesh = plsc.VectorSubcoreMesh(core_axis_name="c", subcore_axis_name="s")
  lpw = _L // _NW

  @functools.partial(
      pl.kernel, mesh=mesh,
      out_type=jax.ShapeDtypeStruct((_R * _L,), jnp.float32),
      scratch_types=[
          pltpu.VMEM((_N,), jnp.float32),
          pltpu.VMEM((_N,), jnp.float32),
          pltpu.VMEM((lpw,), jnp.int32),
          pltpu.VMEM((lpw,), jnp.int32),
          pltpu.VMEM((16,), jnp.float32),
          pltpu.VMEM((16,), jnp.float32),
          pltpu.VMEM((lpw,), jnp.float32),
      ],
      compiler_params=pltpu.CompilerParams(needs_layout_passes=False))
  def body(p0, p1, hidx, tidx, relr, reli, out,
           p0_v, p1_v, hi_v, ti_v, rr_v, ri_v, sc_v):
    cid = lax.axis_index("c")
    sid = lax.axis_index("s")
    wid = cid * _NS + sid
    pltpu.sync_copy(p0, p0_v)
    pltpu.sync_copy(p1, p1_v)
    for r in range(_R):
      pltpu.sync_copy(relr.at[pl.ds(r * 16, 16)], rr_v)
      pltpu.sync_copy(reli.at[pl.ds(r * 16, 16)], ri_v)
      pltpu.sync_copy(hidx.at[pl.ds((r * _NW + wid) * lpw, lpw)], hi_v)
      pltpu.sync_copy(tidx.at[pl.ds((r * _NW + wid) * lpw, lpw)], ti_v)
      rr = rr_v[...]
      ri = ri_v[...]
      for g in range(lpw // 16):
        hvec = hi_v[pl.ds(g * 16, 16)]
        tvec = ti_v[pl.ds(g * 16, 16)]
        h0 = plsc.load_gather(p0_v, [hvec])
        h1 = plsc.load_gather(p1_v, [hvec])
        t0 = plsc.load_gather(p0_v, [tvec])
        t1 = plsc.load_gather(p1_v, [tvec])
        s = rr * (h0 * t0 + h1 * t1) + ri * (h0 * t1 - h1 * t0)
        sc_v[pl.ds(g * 16, 16)] = s
      pltpu.sync_copy(sc_v, out.at[pl.ds((r * _NW + wid) * lpw, lpw)])

  return body


_score = _score_kernel()


def _full(shape):
  return pl.BlockSpec(shape, lambda i: (0,) * len(shape))


def _tc0_body(x_ref, w1_ref, wr1_ref, b1_ref, y1_ref, root1_ref):
  x = x_ref[...]
  for r in range(_R):
    y1_ref[r] = jnp.dot(x, w1_ref[r], preferred_element_type=jnp.float32)
    root1_ref[r] = (jnp.dot(x, wr1_ref[r], preferred_element_type=jnp.float32)
                    + b1_ref[r])


def _gru_block(out_h, past, wi_ref, wh_ref, bi_ref, bh_ref, h):
  gi = jnp.dot(out_h, wi_ref[...], preferred_element_type=jnp.float32) + bi_ref[...]
  gh = jnp.dot(past, wh_ref[...], preferred_element_type=jnp.float32) + bh_ref[...]
  ir, iz, inn = gi[:, :h], gi[:, h:2 * h], gi[:, 2 * h:]
  hr, hz, hn = gh[:, :h], gh[:, h:2 * h], gh[:, 2 * h:]
  rg = jax.nn.sigmoid(ir + hr)
  zg = jax.nn.sigmoid(iz + hz)
  ng = jnp.tanh(inn + rg * hn)
  return (1.0 - zg) * ng + zg * past


def _layer_body(h, snap_ref, aggp_ref, degp_ref, root_ref, past_ref,
                wi_ref, wh_ref, bi_ref, bh_ref, aw_ref, ab_ref, aq_ref,
                cur_ref, wp_ref):
  snap0 = snap_ref[0:1, 0:1]
  lanes = lax.broadcasted_iota(jnp.int32, (8, 128), 1)
  acc = jnp.zeros((8, 128), jnp.float32)
  for r in range(_R):
    agg = aggp_ref[0, r] + aggp_ref[1, r]
    deg = degp_ref[0, r, :, 0:1] + degp_ref[1, r, :, 0:1]
    agg = agg / jnp.maximum(deg, 1.0)
    out_h = jnp.maximum(agg + root_ref[r], 0.0)
    g = _gru_block(out_h, past_ref[r], wi_ref, wh_ref, bi_ref, bh_ref, h)
    cur = jnp.where(snap0 == 0.0, out_h, g)
    cur_ref[r] = cur
    w = jnp.dot(jnp.tanh(jnp.dot(cur, aw_ref[...],
                                 preferred_element_type=jnp.float32)
                         + ab_ref[...]),
                aq_ref[...], preferred_element_type=jnp.float32)
    s = jnp.sum(w[:, 0:1])
    acc = acc + jnp.where(lanes == r, s, 0.0)

  @pl.when(pl.program_id(0) == 0)
  def _init():
    wp_ref[...] = jnp.zeros((8, 128), jnp.float32)

  wp_ref[...] += acc


def _tc2_body(cur1_ref, beta_ref, w2_ref, wr2_ref, b2_ref, y2_ref, root2_ref):
  h1 = cur1_ref[0] * beta_ref[0:1, 0:1]
  h1 = h1 + cur1_ref[1] * beta_ref[1:2, 0:1]
  h1 = h1 + cur1_ref[2] * beta_ref[2:3, 0:1]
  for r in range(_R):
    y2_ref[r] = jnp.dot(h1, w2_ref[r], preferred_element_type=jnp.float32)
    root2_ref[r] = (jnp.dot(h1, wr2_ref[r], preferred_element_type=jnp.float32)
                    + b2_ref[r])


def _tc4_body(cur2_ref, beta_ref, pw_ref, pb_ref, out_ref):
  h2 = cur2_ref[0] * beta_ref[0:1, 0:1]
  h2 = h2 + cur2_ref[1] * beta_ref[1:2, 0:1]
  h2 = h2 + cur2_ref[2] * beta_ref[2:3, 0:1]
  out_ref[...] = (jnp.dot(h2, pw_ref[...], preferred_element_type=jnp.float32)
                  + pb_ref[...])


def kernel(x, edge_index, edge_label_index, snap, past1, past2, W1, Wr1, b1,
           W2, Wr2, b2, g1_Wi, g1_Wh, g1_bi, g1_bh, g2_Wi, g2_Wh, g2_bi,
           g2_bh, a1_W, a1_b, a1_q, a2_W, a2_b, a2_q, post_W, post_b,
           rel_emb):
  f32 = jnp.float32
  snapf = jnp.full((1, 128), snap, f32)
  src = edge_index[:, 0, :].reshape(_R, _NW, _NBATCH, _IB, _CW)
  dst = edge_index[:, 1, :].reshape(_R, _NW, _NBATCH, _IB, _CW)
  zw1 = jnp.zeros((_RPS, _H1), f32)
  zw2 = jnp.zeros((_RPS, _H2), f32)
  z16 = jnp.zeros((_RPS, 16), f32)
  ones = jnp.zeros((_CW, 16), f32).at[:, 0].set(1.0)

  degp = _deg(dst, z16, ones)

  nblk = pl.BlockSpec((_NB, _D), lambda i: (i, 0))

  y1, root1 = pl.pallas_call(
      _tc0_body,
      grid=(_GRID,),
      in_specs=[nblk, _full((_R, _D, _H1)), _full((_R, _D, _H1)),
                _full((_R, 1, _H1))],
      out_specs=[pl.BlockSpec((_R, _NB, _H1), lambda i: (0, i, 0))] * 2,
      out_shape=[jax.ShapeDtypeStruct((_R, _N, _H1), f32)] * 2,
  )(x, W1, Wr1, b1[:, None, :])

  (agg1p,) = _seg128(y1[0], y1[1], y1[2], src, dst, zw1)

  layer1 = functools.partial(_layer_body, _H1)
  cur1, wp1 = pl.pallas_call(
      layer1,
      grid=(_GRID,),
      in_specs=[
          _full((1, 128)),
          pl.BlockSpec((_NC, _R, _NB, _H1), lambda i: (0, 0, i, 0)),
          pl.BlockSpec((_NC, _R, _NB, 16), lambda i: (0, 0, i, 0)),
          pl.BlockSpec((_R, _NB, _H1), lambda i: (0, i, 0)),
          pl.BlockSpec((_R, _NB, _H1), lambda i: (0, i, 0)),
          _full((_H1, 3 * _H1)), _full((_H1, 3 * _H1)),
          _full((1, 3 * _H1)), _full((1, 3 * _H1)),
          _full((_H1, _H1)), _full((1, _H1)), _full((_H1, 8)),
      ],
      out_specs=[pl.BlockSpec((_R, _NB, _H1), lambda i: (0, i, 0)),
                 pl.BlockSpec((8, 128), lambda i: (0, 0))],
      out_shape=[jax.ShapeDtypeStruct((_R, _N, _H1), f32),
                 jax.ShapeDtypeStruct((8, 128), f32)],
  )(snapf, agg1p, degp, root1, past1, g1_Wi, g1_Wh, g1_bi[None, :],
    g1_bh[None, :], a1_W, a1_b[None, :],
    jnp.zeros((_H1, 8), f32).at[:, 0].set(a1_q))

  beta1 = jax.nn.softmax(wp1[0, :_R] / _N)
  beta1b = jnp.broadcast_to(beta1[:, None], (_R, 128))

  y2, root2 = pl.pallas_call(
      _tc2_body,
      grid=(_GRID,),
      in_specs=[pl.BlockSpec((_R, _NB, _H1), lambda i: (0, i, 0)),
                _full((_R, 128)), _full((_R, _H1, _H2)),
                _full((_R, _H1, _H2)), _full((_R, 1, _H2))],
      out_specs=[pl.BlockSpec((_R, _NB, _H2), lambda i: (0, i, 0))] * 2,
      out_shape=[jax.ShapeDtypeStruct((_R, _N, _H2), f32)] * 2,
  )(cur1, beta1b, W2, Wr2, b2[:, None, :])

  (agg2p,) = _seg64(y2[0], y2[1], y2[2], src, dst, zw2)

  layer2 = functools.partial(_layer_body, _H2)
  cur2, wp2 = pl.pallas_call(
      layer2,
      grid=(_GRID,),
      in_specs=[
          _full((1, 128)),
          pl.BlockSpec((_NC, _R, _NB, _H2), lambda i: (0, 0, i, 0)),
          pl.BlockSpec((_NC, _R, _NB, 16), lambda i: (0, 0, i, 0)),
          pl.BlockSpec((_R, _NB, _H2), lambda i: (0, i, 0)),
          pl.BlockSpec((_R, _NB, _H2), lambda i: (0, i, 0)),
          _full((_H2, 3 * _H2)), _full((_H2, 3 * _H2)),
          _full((1, 3 * _H2)), _full((1, 3 * _H2)),
          _full((_H2, _H2)), _full((1, _H2)), _full((_H2, 8)),
      ],
      out_specs=[pl.BlockSpec((_R, _NB, _H2), lambda i: (0, i, 0)),
                 pl.BlockSpec((8, 128), lambda i: (0, 0))],
      out_shape=[jax.ShapeDtypeStruct((_R, _N, _H2), f32),
                 jax.ShapeDtypeStruct((8, 128), f32)],
  )(snapf, agg2p, degp, root2, past2, g2_Wi, g2_Wh, g2_bi[None, :],
    g2_bh[None, :], a2_W, a2_b[None, :],
    jnp.zeros((_H2, 8), f32).at[:, 0].set(a2_q))

  beta2 = jax.nn.softmax(wp2[0, :_R] / _N)
  beta2b = jnp.broadcast_to(beta2[:, None], (_R, 128))

  pwp = jnp.zeros((_H2, 16), f32).at[:, :2].set(post_W)
  pbp = jnp.zeros((1, 16), f32).at[0, :2].set(post_b)
  outp = pl.pallas_call(
      _tc4_body,
      grid=(_GRID,),
      in_specs=[pl.BlockSpec((_R, _NB, _H2), lambda i: (0, i, 0)),
                _full((_R, 128)), _full((_H2, 16)), _full((1, 16))],
      out_specs=pl.BlockSpec((_NB, 16), lambda i: (i, 0)),
      out_shape=jax.ShapeDtypeStruct((_N, 16), f32),
  )(cur2, beta2b, pwp, pbp)

  hidx = edge_label_index[:, 0, :].reshape(-1)
  tidx = edge_label_index[:, 1, :].reshape(-1)
  relr = jnp.broadcast_to(rel_emb[:, 0:1], (_R, 16)).reshape(-1)
  reli = jnp.broadcast_to(rel_emb[:, 1:2], (_R, 16)).reshape(-1)
  scores = _score(outp[:, 0], outp[:, 1], hidx, tidx, relr, reli)

  return scores.reshape(_R, _L), cur1, cur2


# speedup vs baseline: 8.1225x; 1.0397x over previous
import functools

import jax
import jax.numpy as jnp
from jax import lax
from jax.experimental import pallas as pl
from jax.experimental.pallas import tpu as pltpu
from jax.experimental.pallas import tpu_sc as plsc

_N = 10000
_D = 128
_H1 = 128
_H2 = 64
_R = 3
_E = 320000
_L = 8192

_NC = 2
_NS = 16
_NW = _NC * _NS
_EPW = _E // _NW
_CW = 125
_CH = _EPW // _CW
_IB = 10
_NBATCH = _CH // _IB
_NPAD = 10240
_RPS = _NPAD // _NS
_NB = 1000
_GRID = _N // _NB


def _seg_sum_builder(width, with_deg):
  mesh = plsc.VectorSubcoreMesh(core_axis_name="c", subcore_axis_name="s")
  out_type = [jax.ShapeDtypeStruct((_NC, _R, _NPAD, width), jnp.float32)]
  scratch = [
      pltpu.VMEM_SHARED((_NPAD, width), jnp.float32),
      pltpu.VMEM((_IB, _CW), jnp.int32),
      pltpu.VMEM((_IB, _CW), jnp.int32),
      pltpu.VMEM((_CW, width), jnp.float32),
      pltpu.VMEM((_CW, width), jnp.float32),
      pltpu.SemaphoreType.DMA,
      pltpu.SemaphoreType.DMA,
  ]
  if with_deg:
    out_type.append(jax.ShapeDtypeStruct((_NC, _R, _NPAD, 16), jnp.float32))
    scratch += [
        pltpu.VMEM_SHARED((_NPAD, 16), jnp.float32),
        pltpu.VMEM((_CW, 16), jnp.float32),
        pltpu.SemaphoreType.DMA,
    ]

  def body(*refs):
    if with_deg:
      (y0, y1, y2, src, dst, zw, z16, ones, out, outd, acc,
       sidx, didx, rows_a, rows_b, sem_a, sem_b, accd, ones_v, dsem) = refs
    else:
      (y0, y1, y2, src, dst, zw, out, acc,
       sidx, didx, rows_a, rows_b, sem_a, sem_b) = refs
    cid = lax.axis_index("c")
    sid = lax.axis_index("s")
    wid = cid * _NS + sid
    if with_deg:
      pltpu.sync_copy(ones, ones_v)
    for r, yr in enumerate((y0, y1, y2)):
      pltpu.sync_copy(zw, acc.at[pl.ds(sid * _RPS, _RPS)])
      if with_deg:
        pltpu.sync_copy(z16, accd.at[pl.ds(sid * _RPS, _RPS)])
      plsc.subcore_barrier()

      def batch(b, _, yr=yr, r=r):
        pltpu.sync_copy(src.at[r, wid, b], sidx)
        pltpu.sync_copy(dst.at[r, wid, b], didx)
        pltpu.async_copy(yr.at[sidx.at[0]], rows_a, sem_a)

        def pair(p, _):
          c0 = 2 * p
          c1 = c0 + 1
          pltpu.async_copy(yr.at[sidx.at[c1]], rows_b, sem_b)
          pltpu.make_async_copy(yr.at[sidx.at[c0]], rows_a, sem_a).wait()
          pltpu.sync_copy(rows_a, acc.at[didx.at[c0]], add=True)
          if with_deg:
            pltpu.async_copy(ones_v, accd.at[didx.at[c0]], dsem, add=True)

          @pl.when(c1 + 1 < _IB)
          def _prefetch():
            pltpu.async_copy(yr.at[sidx.at[c1 + 1]], rows_a, sem_a)

          pltpu.make_async_copy(yr.at[sidx.at[c1]], rows_b, sem_b).wait()
          pltpu.sync_copy(rows_b, acc.at[didx.at[c1]], add=True)
          if with_deg:
            pltpu.async_copy(ones_v, accd.at[didx.at[c1]], dsem, add=True)
          return _

        res = lax.fori_loop(0, _IB // 2, pair, _)
        if with_deg:
          for _c in range(_IB):
            pltpu.make_async_copy(ones_v, accd.at[didx.at[0]], dsem).wait()
        return res

      lax.fori_loop(0, _NBATCH, batch, None)
      plsc.subcore_barrier()
      pltpu.sync_copy(acc.at[pl.ds(sid * _RPS, _RPS)],
                      out.at[cid, r, pl.ds(sid * _RPS, _RPS)])
      if with_deg:
        pltpu.sync_copy(accd.at[pl.ds(sid * _RPS, _RPS)],
                        outd.at[cid, r, pl.ds(sid * _RPS, _RPS)])
      plsc.subcore_barrier()

  return functools.partial(
      pl.kernel, mesh=mesh, out_type=out_type, scratch_types=scratch,
      compiler_params=pltpu.CompilerParams(use_tc_tiling_on_sc=False))(body)


_seg128 = _seg_sum_builder(_H1, True)
_seg64 = _seg_sum_builder(_H2, False)


def _score_kernel():
  mesh = plsc.VectorSubcoreMesh(core_axis_name="c", subcore_axis_name="s")
  lpw = _L // _NW

  @functools.partial(
      pl.kernel, mesh=mesh,
      out_type=jax.ShapeDtypeStruct((_R * _L,), jnp.float32),
      scratch_types=[
          pltpu.VMEM((_N,), jnp.float32),
          pltpu.VMEM((_N,), jnp.float32),
          pltpu.VMEM((lpw,), jnp.int32),
          pltpu.VMEM((lpw,), jnp.int32),
          pltpu.VMEM((16,), jnp.float32),
          pltpu.VMEM((16,), jnp.float32),
          pltpu.VMEM((lpw,), jnp.float32),
      ],
      compiler_params=pltpu.CompilerParams(needs_layout_passes=False))
  def body(p0, p1, hidx, tidx, relr, reli, out,
           p0_v, p1_v, hi_v, ti_v, rr_v, ri_v, sc_v):
    cid = lax.axis_index("c")
    sid = lax.axis_index("s")
    wid = cid * _NS + sid
    pltpu.sync_copy(p0, p0_v)
    pltpu.sync_copy(p1, p1_v)
    for r in range(_R):
      pltpu.sync_copy(relr.at[pl.ds(r * 16, 16)], rr_v)
      pltpu.sync_copy(reli.at[pl.ds(r * 16, 16)], ri_v)
      pltpu.sync_copy(hidx.at[pl.ds((r * _NW + wid) * lpw, lpw)], hi_v)
      pltpu.sync_copy(tidx.at[pl.ds((r * _NW + wid) * lpw, lpw)], ti_v)
      rr = rr_v[...]
      ri = ri_v[...]
      for g in range(lpw // 16):
        hvec = hi_v[pl.ds(g * 16, 16)]
        tvec = ti_v[pl.ds(g * 16, 16)]
        h0 = plsc.load_gather(p0_v, [hvec])
        h1 = plsc.load_gather(p1_v, [hvec])
        t0 = plsc.load_gather(p0_v, [tvec])
        t1 = plsc.load_gather(p1_v, [tvec])
        s = rr * (h0 * t0 + h1 * t1) + ri * (h0 * t1 - h1 * t0)
        sc_v[pl.ds(g * 16, 16)] = s
      pltpu.sync_copy(sc_v, out.at[pl.ds((r * _NW + wid) * lpw, lpw)])

  return body


_score = _score_kernel()


def _full(shape):
  return pl.BlockSpec(shape, lambda i: (0,) * len(shape))


def _tc0_body(x_ref, w1_ref, wr1_ref, b1_ref, y1_ref, root1_ref):
  x = x_ref[...]
  for r in range(_R):
    y1_ref[r] = jnp.dot(x, w1_ref[r], preferred_element_type=jnp.float32)
    root1_ref[r] = (jnp.dot(x, wr1_ref[r], preferred_element_type=jnp.float32)
                    + b1_ref[r])


def _gru_block(out_h, past, wi_ref, wh_ref, bi_ref, bh_ref, h):
  gi = jnp.dot(out_h, wi_ref[...], preferred_element_type=jnp.float32) + bi_ref[...]
  gh = jnp.dot(past, wh_ref[...], preferred_element_type=jnp.float32) + bh_ref[...]
  ir, iz, inn = gi[:, :h], gi[:, h:2 * h], gi[:, 2 * h:]
  hr, hz, hn = gh[:, :h], gh[:, h:2 * h], gh[:, 2 * h:]
  rg = jax.nn.sigmoid(ir + hr)
  zg = jax.nn.sigmoid(iz + hz)
  ng = jnp.tanh(inn + rg * hn)
  return (1.0 - zg) * ng + zg * past


def _layer_body(h, snap_ref, aggp_ref, degp_ref, root_ref, past_ref,
                wi_ref, wh_ref, bi_ref, bh_ref, aw_ref, ab_ref, aq_ref,
                cur_ref, wp_ref):
  snap0 = snap_ref[0:1, 0:1]
  lanes = lax.broadcasted_iota(jnp.int32, (8, 128), 1)
  acc = jnp.zeros((8, 128), jnp.float32)
  for r in range(_R):
    agg = aggp_ref[0, r] + aggp_ref[1, r]
    deg = degp_ref[0, r, :, 0:1] + degp_ref[1, r, :, 0:1]
    agg = agg / jnp.maximum(deg, 1.0)
    out_h = jnp.maximum(agg + root_ref[r], 0.0)
    g = _gru_block(out_h, past_ref[r], wi_ref, wh_ref, bi_ref, bh_ref, h)
    cur = jnp.where(snap0 == 0.0, out_h, g)
    cur_ref[r] = cur
    w = jnp.dot(jnp.tanh(jnp.dot(cur, aw_ref[...],
                                 preferred_element_type=jnp.float32)
                         + ab_ref[...]),
                aq_ref[...], preferred_element_type=jnp.float32)
    s = jnp.sum(w[:, 0:1])
    acc = acc + jnp.where(lanes == r, s, 0.0)

  @pl.when(pl.program_id(0) == 0)
  def _init():
    wp_ref[...] = jnp.zeros((8, 128), jnp.float32)

  wp_ref[...] += acc


def _tc2_body(cur1_ref, beta_ref, w2_ref, wr2_ref, b2_ref, y2_ref, root2_ref):
  h1 = cur1_ref[0] * beta_ref[0:1, 0:1]
  h1 = h1 + cur1_ref[1] * beta_ref[1:2, 0:1]
  h1 = h1 + cur1_ref[2] * beta_ref[2:3, 0:1]
  for r in range(_R):
    y2_ref[r] = jnp.dot(h1, w2_ref[r], preferred_element_type=jnp.float32)
    root2_ref[r] = (jnp.dot(h1, wr2_ref[r], preferred_element_type=jnp.float32)
                    + b2_ref[r])


def _tc4_body(cur2_ref, beta_ref, pw_ref, pb_ref, out_ref):
  h2 = cur2_ref[0] * beta_ref[0:1, 0:1]
  h2 = h2 + cur2_ref[1] * beta_ref[1:2, 0:1]
  h2 = h2 + cur2_ref[2] * beta_ref[2:3, 0:1]
  out_ref[...] = (jnp.dot(h2, pw_ref[...], preferred_element_type=jnp.float32)
                  + pb_ref[...])


def kernel(x, edge_index, edge_label_index, snap, past1, past2, W1, Wr1, b1,
           W2, Wr2, b2, g1_Wi, g1_Wh, g1_bi, g1_bh, g2_Wi, g2_Wh, g2_bi,
           g2_bh, a1_W, a1_b, a1_q, a2_W, a2_b, a2_q, post_W, post_b,
           rel_emb):
  f32 = jnp.float32
  snapf = jnp.full((1, 128), snap, f32)
  src = edge_index[:, 0, :].reshape(_R, _NW, _NBATCH, _IB, _CW)
  dst = edge_index[:, 1, :].reshape(_R, _NW, _NBATCH, _IB, _CW)
  zw1 = jnp.zeros((_RPS, _H1), f32)
  zw2 = jnp.zeros((_RPS, _H2), f32)
  z16 = jnp.zeros((_RPS, 16), f32)
  ones = jnp.zeros((_CW, 16), f32).at[:, 0].set(1.0)

  nblk = pl.BlockSpec((_NB, _D), lambda i: (i, 0))

  y1, root1 = pl.pallas_call(
      _tc0_body,
      grid=(_GRID,),
      in_specs=[nblk, _full((_R, _D, _H1)), _full((_R, _D, _H1)),
                _full((_R, 1, _H1))],
      out_specs=[pl.BlockSpec((_R, _NB, _H1), lambda i: (0, i, 0))] * 2,
      out_shape=[jax.ShapeDtypeStruct((_R, _N, _H1), f32)] * 2,
  )(x, W1, Wr1, b1[:, None, :])

  agg1p, degp = _seg128(y1[0], y1[1], y1[2], src, dst, zw1, z16, ones)

  layer1 = functools.partial(_layer_body, _H1)
  cur1, wp1 = pl.pallas_call(
      layer1,
      grid=(_GRID,),
      in_specs=[
          _full((1, 128)),
          pl.BlockSpec((_NC, _R, _NB, _H1), lambda i: (0, 0, i, 0)),
          pl.BlockSpec((_NC, _R, _NB, 16), lambda i: (0, 0, i, 0)),
          pl.BlockSpec((_R, _NB, _H1), lambda i: (0, i, 0)),
          pl.BlockSpec((_R, _NB, _H1), lambda i: (0, i, 0)),
          _full((_H1, 3 * _H1)), _full((_H1, 3 * _H1)),
          _full((1, 3 * _H1)), _full((1, 3 * _H1)),
          _full((_H1, _H1)), _full((1, _H1)), _full((_H1, 8)),
      ],
      out_specs=[pl.BlockSpec((_R, _NB, _H1), lambda i: (0, i, 0)),
                 pl.BlockSpec((8, 128), lambda i: (0, 0))],
      out_shape=[jax.ShapeDtypeStruct((_R, _N, _H1), f32),
                 jax.ShapeDtypeStruct((8, 128), f32)],
  )(snapf, agg1p, degp, root1, past1, g1_Wi, g1_Wh, g1_bi[None, :],
    g1_bh[None, :], a1_W, a1_b[None, :],
    jnp.zeros((_H1, 8), f32).at[:, 0].set(a1_q))

  beta1 = jax.nn.softmax(wp1[0, :_R] / _N)
  beta1b = jnp.broadcast_to(beta1[:, None], (_R, 128))

  y2, root2 = pl.pallas_call(
      _tc2_body,
      grid=(_GRID,),
      in_specs=[pl.BlockSpec((_R, _NB, _H1), lambda i: (0, i, 0)),
                _full((_R, 128)), _full((_R, _H1, _H2)),
                _full((_R, _H1, _H2)), _full((_R, 1, _H2))],
      out_specs=[pl.BlockSpec((_R, _NB, _H2), lambda i: (0, i, 0))] * 2,
      out_shape=[jax.ShapeDtypeStruct((_R, _N, _H2), f32)] * 2,
  )(cur1, beta1b, W2, Wr2, b2[:, None, :])

  (agg2p,) = _seg64(y2[0], y2[1], y2[2], src, dst, zw2)

  layer2 = functools.partial(_layer_body, _H2)
  cur2, wp2 = pl.pallas_call(
      layer2,
      grid=(_GRID,),
      in_specs=[
          _full((1, 128)),
          pl.BlockSpec((_NC, _R, _NB, _H2), lambda i: (0, 0, i, 0)),
          pl.BlockSpec((_NC, _R, _NB, 16), lambda i: (0, 0, i, 0)),
          pl.BlockSpec((_R, _NB, _H2), lambda i: (0, i, 0)),
          pl.BlockSpec((_R, _NB, _H2), lambda i: (0, i, 0)),
          _full((_H2, 3 * _H2)), _full((_H2, 3 * _H2)),
          _full((1, 3 * _H2)), _full((1, 3 * _H2)),
          _full((_H2, _H2)), _full((1, _H2)), _full((_H2, 8)),
      ],
      out_specs=[pl.BlockSpec((_R, _NB, _H2), lambda i: (0, i, 0)),
                 pl.BlockSpec((8, 128), lambda i: (0, 0))],
      out_shape=[jax.ShapeDtypeStruct((_R, _N, _H2), f32),
                 jax.ShapeDtypeStruct((8, 128), f32)],
  )(snapf, agg2p, degp, root2, past2, g2_Wi, g2_Wh, g2_bi[None, :],
    g2_bh[None, :], a2_W, a2_b[None, :],
    jnp.zeros((_H2, 8), f32).at[:, 0].set(a2_q))

  beta2 = jax.nn.softmax(wp2[0, :_R] / _N)
  beta2b = jnp.broadcast_to(beta2[:, None], (_R, 128))

  pwp = jnp.zeros((_H2, 16), f32).at[:, :2].set(post_W)
  pbp = jnp.zeros((1, 16), f32).at[0, :2].set(post_b)
  outp = pl.pallas_call(
      _tc4_body,
      grid=(_GRID,),
      in_specs=[pl.BlockSpec((_R, _NB, _H2), lambda i: (0, i, 0)),
                _full((_R, 128)), _full((_H2, 16)), _full((1, 16))],
      out_specs=pl.BlockSpec((_NB, 16), lambda i: (i, 0)),
      out_shape=jax.ShapeDtypeStruct((_N, 16), f32),
  )(cur2, beta2b, pwp, pbp)

  hidx = edge_label_index[:, 0, :].reshape(-1)
  tidx = edge_label_index[:, 1, :].reshape(-1)
  relr = jnp.broadcast_to(rel_emb[:, 0:1], (_R, 16)).reshape(-1)
  reli = jnp.broadcast_to(rel_emb[:, 1:2], (_R, 16)).reshape(-1)
  scores = _score(outp[:, 0], outp[:, 1], hidx, tidx, relr, reli)

  return scores.reshape(_R, _L), cur1, cur2


# speedup vs baseline: 8.2725x; 1.0185x over previous
import functools

import jax
import jax.numpy as jnp
from jax import lax
from jax.experimental import pallas as pl
from jax.experimental.pallas import tpu as pltpu
from jax.experimental.pallas import tpu_sc as plsc

_N = 10000
_D = 128
_H1 = 128
_H2 = 64
_R = 3
_E = 320000
_L = 8192

_NC = 2
_NS = 16
_NW = _NC * _NS
_EPW = _E // _NW
_CW = 125
_CH = _EPW // _CW
_IB = 10
_NBATCH = _CH // _IB
_NPAD = 10240
_RPS = _NPAD // _NS
_NB = 1000
_GRID = _N // _NB


def _seg_sum_builder(width, with_deg):
  mesh = plsc.VectorSubcoreMesh(core_axis_name="c", subcore_axis_name="s")
  out_type = [jax.ShapeDtypeStruct((_NC, _R, _NPAD, width), jnp.float32)]
  scratch = [
      pltpu.VMEM_SHARED((_NPAD, width), jnp.float32),
      pltpu.VMEM((_IB, _CW), jnp.int32),
      pltpu.VMEM((_IB, _CW), jnp.int32),
      pltpu.VMEM((_CW, width), jnp.float32),
      pltpu.VMEM((_CW, width), jnp.float32),
      pltpu.SemaphoreType.DMA,
      pltpu.SemaphoreType.DMA,
  ]
  if with_deg:
    out_type.append(jax.ShapeDtypeStruct((_NC, _R, _NPAD, 16), jnp.float32))
    scratch += [
        pltpu.VMEM_SHARED((_NPAD, 16), jnp.float32),
        pltpu.VMEM((_CW, 16), jnp.float32),
        pltpu.SemaphoreType.DMA,
    ]

  def body(*refs):
    if with_deg:
      (y0, y1, y2, src, dst, zw, z16, ones, out, outd, acc,
       sidx, didx, rows_a, rows_b, sem_a, sem_b, accd, ones_v, dsem) = refs
    else:
      (y0, y1, y2, src, dst, zw, out, acc,
       sidx, didx, rows_a, rows_b, sem_a, sem_b) = refs
    cid = lax.axis_index("c")
    sid = lax.axis_index("s")
    wid = cid * _NS + sid
    if with_deg:
      pltpu.sync_copy(ones, ones_v)
    for r, yr in enumerate((y0, y1, y2)):
      pltpu.sync_copy(zw, acc.at[pl.ds(sid * _RPS, _RPS)])
      if with_deg:
        pltpu.sync_copy(z16, accd.at[pl.ds(sid * _RPS, _RPS)])
      plsc.subcore_barrier()

      def batch(b, _, yr=yr, r=r):
        pltpu.sync_copy(src.at[r, wid, b], sidx)
        pltpu.sync_copy(dst.at[r, wid, b], didx)
        pltpu.async_copy(yr.at[sidx.at[0]], rows_a, sem_a)

        def pair(p, _):
          c0 = 2 * p
          c1 = c0 + 1
          pltpu.async_copy(yr.at[sidx.at[c1]], rows_b, sem_b)
          pltpu.make_async_copy(yr.at[sidx.at[c0]], rows_a, sem_a).wait()
          pltpu.sync_copy(rows_a, acc.at[didx.at[c0]], add=True)
          if with_deg:
            pltpu.async_copy(ones_v, accd.at[didx.at[c0]], dsem, add=True)

          @pl.when(c1 + 1 < _IB)
          def _prefetch():
            pltpu.async_copy(yr.at[sidx.at[c1 + 1]], rows_a, sem_a)

          pltpu.make_async_copy(yr.at[sidx.at[c1]], rows_b, sem_b).wait()
          pltpu.sync_copy(rows_b, acc.at[didx.at[c1]], add=True)
          if with_deg:
            pltpu.async_copy(ones_v, accd.at[didx.at[c1]], dsem, add=True)
          return _

        res = lax.fori_loop(0, _IB // 2, pair, _)
        if with_deg:
          for _c in range(_IB):
            pltpu.make_async_copy(ones_v, accd.at[didx.at[0]], dsem).wait()
        return res

      lax.fori_loop(0, _NBATCH, batch, None)
      plsc.subcore_barrier()
      pltpu.sync_copy(acc.at[pl.ds(sid * _RPS, _RPS)],
                      out.at[cid, r, pl.ds(sid * _RPS, _RPS)])
      if with_deg:
        pltpu.sync_copy(accd.at[pl.ds(sid * _RPS, _RPS)],
                        outd.at[cid, r, pl.ds(sid * _RPS, _RPS)])
      plsc.subcore_barrier()

  return functools.partial(
      pl.kernel, mesh=mesh, out_type=out_type, scratch_types=scratch,
      compiler_params=pltpu.CompilerParams(use_tc_tiling_on_sc=False))(body)


_seg128 = _seg_sum_builder(_H1, True)
_seg64 = _seg_sum_builder(_H2, False)


def _score_kernel():
  mesh = plsc.VectorSubcoreMesh(core_axis_name="c", subcore_axis_name="s")
  lpw = _L // _NW

  @functools.partial(
      pl.kernel, mesh=mesh,
      out_type=jax.ShapeDtypeStruct((_R * _L,), jnp.float32),
      scratch_types=[
          pltpu.VMEM((_N,), jnp.float32),
          pltpu.VMEM((_N,), jnp.float32),
          pltpu.VMEM((lpw,), jnp.int32),
          pltpu.VMEM((lpw,), jnp.int32),
          pltpu.VMEM((16,), jnp.float32),
          pltpu.VMEM((16,), jnp.float32),
          pltpu.VMEM((lpw,), jnp.float32),
      ],
      compiler_params=pltpu.CompilerParams(needs_layout_passes=False))
  def body(p0, p1, hidx, tidx, relr, reli, out,
           p0_v, p1_v, hi_v, ti_v, rr_v, ri_v, sc_v):
    cid = lax.axis_index("c")
    sid = lax.axis_index("s")
    wid = cid * _NS + sid
    pltpu.sync_copy(p0, p0_v)
    pltpu.sync_copy(p1, p1_v)
    for r in range(_R):
      pltpu.sync_copy(relr.at[pl.ds(r * 16, 16)], rr_v)
      pltpu.sync_copy(reli.at[pl.ds(r * 16, 16)], ri_v)
      pltpu.sync_copy(hidx.at[pl.ds((r * _NW + wid) * lpw, lpw)], hi_v)
      pltpu.sync_copy(tidx.at[pl.ds((r * _NW + wid) * lpw, lpw)], ti_v)
      rr = rr_v[...]
      ri = ri_v[...]
      for g in range(lpw // 16):
        hvec = hi_v[pl.ds(g * 16, 16)]
        tvec = ti_v[pl.ds(g * 16, 16)]
        h0 = plsc.load_gather(p0_v, [hvec])
        h1 = plsc.load_gather(p1_v, [hvec])
        t0 = plsc.load_gather(p0_v, [tvec])
        t1 = plsc.load_gather(p1_v, [tvec])
        s = rr * (h0 * t0 + h1 * t1) + ri * (h0 * t1 - h1 * t0)
        sc_v[pl.ds(g * 16, 16)] = s
      pltpu.sync_copy(sc_v, out.at[pl.ds((r * _NW + wid) * lpw, lpw)])

  return body


_score = _score_kernel()


def _full(shape):
  return pl.BlockSpec(shape, lambda i: (0,) * len(shape))


def _tc0_body(x_ref, w1_ref, wr1_ref, b1_ref, y1a_ref, y1b_ref, y1c_ref,
              root1_ref):
  x = x_ref[...]
  for r, yref in enumerate((y1a_ref, y1b_ref, y1c_ref)):
    yref[...] = jnp.dot(x, w1_ref[r], preferred_element_type=jnp.float32)
    root1_ref[r] = (jnp.dot(x, wr1_ref[r], preferred_element_type=jnp.float32)
                    + b1_ref[r])


def _gru_block(out_h, past, wi_ref, wh_ref, bi_ref, bh_ref, h):
  gi = jnp.dot(out_h, wi_ref[...], preferred_element_type=jnp.float32) + bi_ref[...]
  gh = jnp.dot(past, wh_ref[...], preferred_element_type=jnp.float32) + bh_ref[...]
  ir, iz, inn = gi[:, :h], gi[:, h:2 * h], gi[:, 2 * h:]
  hr, hz, hn = gh[:, :h], gh[:, h:2 * h], gh[:, 2 * h:]
  rg = jax.nn.sigmoid(ir + hr)
  zg = jax.nn.sigmoid(iz + hz)
  ng = jnp.tanh(inn + rg * hn)
  return (1.0 - zg) * ng + zg * past


def _layer_body(h, snap_ref, aggp_ref, degp_ref, root_ref, past_ref,
                wi_ref, wh_ref, bi_ref, bh_ref, aw_ref, ab_ref, aq_ref,
                cur_ref, wp_ref):
  snap0 = snap_ref[0:1, 0:1]
  lanes = lax.broadcasted_iota(jnp.int32, (8, 128), 1)
  acc = jnp.zeros((8, 128), jnp.float32)
  for r in range(_R):
    agg = aggp_ref[0, r] + aggp_ref[1, r]
    deg = degp_ref[0, r, :, 0:1] + degp_ref[1, r, :, 0:1]
    agg = agg / jnp.maximum(deg, 1.0)
    out_h = jnp.maximum(agg + root_ref[r], 0.0)
    g = _gru_block(out_h, past_ref[r], wi_ref, wh_ref, bi_ref, bh_ref, h)
    cur = jnp.where(snap0 == 0.0, out_h, g)
    cur_ref[r] = cur
    w = jnp.dot(jnp.tanh(jnp.dot(cur, aw_ref[...],
                                 preferred_element_type=jnp.float32)
                         + ab_ref[...]),
                aq_ref[...], preferred_element_type=jnp.float32)
    s = jnp.sum(w[:, 0:1])
    acc = acc + jnp.where(lanes == r, s, 0.0)

  @pl.when(pl.program_id(0) == 0)
  def _init():
    wp_ref[...] = jnp.zeros((8, 128), jnp.float32)

  wp_ref[...] += acc


def _tc2_body(cur1_ref, beta_ref, w2_ref, wr2_ref, b2_ref,
              y2a_ref, y2b_ref, y2c_ref, root2_ref):
  h1 = cur1_ref[0] * beta_ref[0:1, 0:1]
  h1 = h1 + cur1_ref[1] * beta_ref[1:2, 0:1]
  h1 = h1 + cur1_ref[2] * beta_ref[2:3, 0:1]
  for r, yref in enumerate((y2a_ref, y2b_ref, y2c_ref)):
    yref[...] = jnp.dot(h1, w2_ref[r], preferred_element_type=jnp.float32)
    root2_ref[r] = (jnp.dot(h1, wr2_ref[r], preferred_element_type=jnp.float32)
                    + b2_ref[r])


def _tc4_body(cur2_ref, beta_ref, pw_ref, pb_ref, out_ref):
  h2 = cur2_ref[0] * beta_ref[0:1, 0:1]
  h2 = h2 + cur2_ref[1] * beta_ref[1:2, 0:1]
  h2 = h2 + cur2_ref[2] * beta_ref[2:3, 0:1]
  out_ref[...] = (jnp.dot(h2, pw_ref[...], preferred_element_type=jnp.float32)
                  + pb_ref[...])


def kernel(x, edge_index, edge_label_index, snap, past1, past2, W1, Wr1, b1,
           W2, Wr2, b2, g1_Wi, g1_Wh, g1_bi, g1_bh, g2_Wi, g2_Wh, g2_bi,
           g2_bh, a1_W, a1_b, a1_q, a2_W, a2_b, a2_q, post_W, post_b,
           rel_emb):
  f32 = jnp.float32
  snapf = jnp.full((1, 128), snap, f32)
  src = edge_index[:, 0, :].reshape(_R, _NW, _NBATCH, _IB, _CW)
  dst = edge_index[:, 1, :].reshape(_R, _NW, _NBATCH, _IB, _CW)
  zw1 = jnp.zeros((_RPS, _H1), f32)
  zw2 = jnp.zeros((_RPS, _H2), f32)
  z16 = jnp.zeros((_RPS, 16), f32)
  ones = jnp.zeros((_CW, 16), f32).at[:, 0].set(1.0)

  nblk = pl.BlockSpec((_NB, _D), lambda i: (i, 0))

  y1a, y1b, y1c, root1 = pl.pallas_call(
      _tc0_body,
      grid=(_GRID,),
      in_specs=[nblk, _full((_R, _D, _H1)), _full((_R, _D, _H1)),
                _full((_R, 1, _H1))],
      out_specs=[nblk] * 3 + [pl.BlockSpec((_R, _NB, _H1), lambda i: (0, i, 0))],
      out_shape=[jax.ShapeDtypeStruct((_N, _H1), f32)] * 3
      + [jax.ShapeDtypeStruct((_R, _N, _H1), f32)],
  )(x, W1, Wr1, b1[:, None, :])

  agg1p, degp = _seg128(y1a, y1b, y1c, src, dst, zw1, z16, ones)

  layer1 = functools.partial(_layer_body, _H1)
  cur1, wp1 = pl.pallas_call(
      layer1,
      grid=(_GRID,),
      in_specs=[
          _full((1, 128)),
          pl.BlockSpec((_NC, _R, _NB, _H1), lambda i: (0, 0, i, 0)),
          pl.BlockSpec((_NC, _R, _NB, 16), lambda i: (0, 0, i, 0)),
          pl.BlockSpec((_R, _NB, _H1), lambda i: (0, i, 0)),
          pl.BlockSpec((_R, _NB, _H1), lambda i: (0, i, 0)),
          _full((_H1, 3 * _H1)), _full((_H1, 3 * _H1)),
          _full((1, 3 * _H1)), _full((1, 3 * _H1)),
          _full((_H1, _H1)), _full((1, _H1)), _full((_H1, 8)),
      ],
      out_specs=[pl.BlockSpec((_R, _NB, _H1), lambda i: (0, i, 0)),
                 pl.BlockSpec((8, 128), lambda i: (0, 0))],
      out_shape=[jax.ShapeDtypeStruct((_R, _N, _H1), f32),
                 jax.ShapeDtypeStruct((8, 128), f32)],
  )(snapf, agg1p, degp, root1, past1, g1_Wi, g1_Wh, g1_bi[None, :],
    g1_bh[None, :], a1_W, a1_b[None, :],
    jnp.zeros((_H1, 8), f32).at[:, 0].set(a1_q))

  beta1 = jax.nn.softmax(wp1[0, :_R] / _N)
  beta1b = jnp.broadcast_to(beta1[:, None], (_R, 128))

  h2blk = pl.BlockSpec((_NB, _H2), lambda i: (i, 0))
  y2a, y2b, y2c, root2 = pl.pallas_call(
      _tc2_body,
      grid=(_GRID,),
      in_specs=[pl.BlockSpec((_R, _NB, _H1), lambda i: (0, i, 0)),
                _full((_R, 128)), _full((_R, _H1, _H2)),
                _full((_R, _H1, _H2)), _full((_R, 1, _H2))],
      out_specs=[h2blk] * 3 + [pl.BlockSpec((_R, _NB, _H2), lambda i: (0, i, 0))],
      out_shape=[jax.ShapeDtypeStruct((_N, _H2), f32)] * 3
      + [jax.ShapeDtypeStruct((_R, _N, _H2), f32)],
  )(cur1, beta1b, W2, Wr2, b2[:, None, :])

  (agg2p,) = _seg64(y2a, y2b, y2c, src, dst, zw2)

  layer2 = functools.partial(_layer_body, _H2)
  cur2, wp2 = pl.pallas_call(
      layer2,
      grid=(_GRID,),
      in_specs=[
          _full((1, 128)),
          pl.BlockSpec((_NC, _R, _NB, _H2), lambda i: (0, 0, i, 0)),
          pl.BlockSpec((_NC, _R, _NB, 16), lambda i: (0, 0, i, 0)),
          pl.BlockSpec((_R, _NB, _H2), lambda i: (0, i, 0)),
          pl.BlockSpec((_R, _NB, _H2), lambda i: (0, i, 0)),
          _full((_H2, 3 * _H2)), _full((_H2, 3 * _H2)),
          _full((1, 3 * _H2)), _full((1, 3 * _H2)),
          _full((_H2, _H2)), _full((1, _H2)), _full((_H2, 8)),
      ],
      out_specs=[pl.BlockSpec((_R, _NB, _H2), lambda i: (0, i, 0)),
                 pl.BlockSpec((8, 128), lambda i: (0, 0))],
      out_shape=[jax.ShapeDtypeStruct((_R, _N, _H2), f32),
                 jax.ShapeDtypeStruct((8, 128), f32)],
  )(snapf, agg2p, degp, root2, past2, g2_Wi, g2_Wh, g2_bi[None, :],
    g2_bh[None, :], a2_W, a2_b[None, :],
    jnp.zeros((_H2, 8), f32).at[:, 0].set(a2_q))

  beta2 = jax.nn.softmax(wp2[0, :_R] / _N)
  beta2b = jnp.broadcast_to(beta2[:, None], (_R, 128))

  pwp = jnp.zeros((_H2, 16), f32).at[:, :2].set(post_W)
  pbp = jnp.zeros((1, 16), f32).at[0, :2].set(post_b)
  outp = pl.pallas_call(
      _tc4_body,
      grid=(_GRID,),
      in_specs=[pl.BlockSpec((_R, _NB, _H2), lambda i: (0, i, 0)),
                _full((_R, 128)), _full((_H2, 16)), _full((1, 16))],
      out_specs=pl.BlockSpec((_NB, 16), lambda i: (i, 0)),
      out_shape=jax.ShapeDtypeStruct((_N, 16), f32),
  )(cur2, beta2b, pwp, pbp)

  hidx = edge_label_index[:, 0, :].reshape(-1)
  tidx = edge_label_index[:, 1, :].reshape(-1)
  relr = jnp.broadcast_to(rel_emb[:, 0:1], (_R, 16)).reshape(-1)
  reli = jnp.broadcast_to(rel_emb[:, 1:2], (_R, 16)).reshape(-1)
  scores = _score(outp[:, 0], outp[:, 1], hidx, tidx, relr, reli)

  return scores.reshape(_R, _L), cur1, cur2


# speedup vs baseline: 8.5557x; 1.0342x over previous
import functools

import jax
import jax.numpy as jnp
from jax import lax
from jax.experimental import pallas as pl
from jax.experimental.pallas import tpu as pltpu
from jax.experimental.pallas import tpu_sc as plsc

_N = 10000
_D = 128
_H1 = 128
_H2 = 64
_R = 3
_E = 320000
_L = 8192

_NC = 2
_NS = 16
_NW = _NC * _NS
_EPW = _E // _NW
_CW = 125
_CH = _EPW // _CW
_IB = 10
_NBATCH = _CH // _IB
_NPAD = 10240
_RPS = _NPAD // _NS
_NB = 1000
_GRID = _N // _NB


def _seg_sum_builder(width, with_deg, nbuf=2):
  mesh = plsc.VectorSubcoreMesh(core_axis_name="c", subcore_axis_name="s")
  out_type = [jax.ShapeDtypeStruct((_NC, _R, _NPAD, width), jnp.float32)]
  scratch = [
      pltpu.VMEM_SHARED((_NPAD, width), jnp.float32),
      pltpu.VMEM((_IB, _CW), jnp.int32),
      pltpu.VMEM((_IB, _CW), jnp.int32),
  ]
  scratch += [pltpu.VMEM((_CW, width), jnp.float32)] * nbuf
  scratch += [pltpu.SemaphoreType.DMA] * nbuf
  if nbuf > 2:
    scratch += [pltpu.SemaphoreType.DMA] * nbuf
  if with_deg:
    out_type.append(jax.ShapeDtypeStruct((_NC, _R, _NPAD, 16), jnp.float32))
    scratch += [
        pltpu.VMEM_SHARED((_NPAD, 16), jnp.float32),
        pltpu.VMEM((_CW, 16), jnp.float32),
        pltpu.SemaphoreType.DMA,
    ]

  def body(*refs):
    y0, y1, y2, src, dst, zw = refs[:6]
    k = 6
    if with_deg:
      z16, ones = refs[k:k + 2]
      k += 2
    out = refs[k]
    k += 1
    if with_deg:
      outd = refs[k]
      k += 1
    acc, sidx, didx = refs[k:k + 3]
    k += 3
    rows = refs[k:k + nbuf]
    k += nbuf
    gsem = refs[k:k + nbuf]
    k += nbuf
    if nbuf > 2:
      ssem = refs[k:k + nbuf]
      k += nbuf
    if with_deg:
      accd, ones_v, dsem = refs[k:k + 3]
    cid = lax.axis_index("c")
    sid = lax.axis_index("s")
    wid = cid * _NS + sid
    if with_deg:
      pltpu.sync_copy(ones, ones_v)
    for r, yr in enumerate((y0, y1, y2)):
      pltpu.sync_copy(zw, acc.at[pl.ds(sid * _RPS, _RPS)])
      if with_deg:
        pltpu.sync_copy(z16, accd.at[pl.ds(sid * _RPS, _RPS)])
      plsc.subcore_barrier()

      if nbuf == 2:
        def batch(b, _, yr=yr, r=r):
          pltpu.sync_copy(src.at[r, wid, b], sidx)
          pltpu.sync_copy(dst.at[r, wid, b], didx)
          pltpu.async_copy(yr.at[sidx.at[0]], rows[0], gsem[0])

          def pair(p, _):
            c0 = 2 * p
            c1 = c0 + 1
            pltpu.async_copy(yr.at[sidx.at[c1]], rows[1], gsem[1])
            pltpu.make_async_copy(yr.at[sidx.at[c0]], rows[0], gsem[0]).wait()
            pltpu.sync_copy(rows[0], acc.at[didx.at[c0]], add=True)
            if with_deg:
              pltpu.async_copy(ones_v, accd.at[didx.at[c0]], dsem, add=True)

            @pl.when(c1 + 1 < _IB)
            def _prefetch():
              pltpu.async_copy(yr.at[sidx.at[c1 + 1]], rows[0], gsem[0])

            pltpu.make_async_copy(yr.at[sidx.at[c1]], rows[1], gsem[1]).wait()
            pltpu.sync_copy(rows[1], acc.at[didx.at[c1]], add=True)
            if with_deg:
              pltpu.async_copy(ones_v, accd.at[didx.at[c1]], dsem, add=True)
            return _

          res = lax.fori_loop(0, _IB // 2, pair, _)
          if with_deg:
            for _c in range(_IB):
              pltpu.make_async_copy(ones_v, accd.at[didx.at[0]], dsem).wait()
          return res
      else:
        def batch(b, _, yr=yr, r=r):
          pltpu.sync_copy(src.at[r, wid, b], sidx)
          pltpu.sync_copy(dst.at[r, wid, b], didx)
          pltpu.async_copy(yr.at[sidx.at[0]], rows[0], gsem[0])
          pltpu.async_copy(yr.at[sidx.at[1]], rows[1], gsem[1])
          for c in range(_IB):
            nxt = c + 2
            if nxt < _IB:
              j = nxt % nbuf
              if nxt >= nbuf:
                pltpu.make_async_copy(
                    rows[j], acc.at[didx.at[0]], ssem[j]).wait()
              pltpu.async_copy(yr.at[sidx.at[nxt]], rows[j], gsem[j])
            i = c % nbuf
            pltpu.make_async_copy(yr.at[sidx.at[c]], rows[i], gsem[i]).wait()
            pltpu.async_copy(rows[i], acc.at[didx.at[c]], ssem[i], add=True)
          for j in range(nbuf):
            pltpu.make_async_copy(rows[j], acc.at[didx.at[0]], ssem[j]).wait()
          return _

      lax.fori_loop(0, _NBATCH, batch, None)
      plsc.subcore_barrier()
      pltpu.sync_copy(acc.at[pl.ds(sid * _RPS, _RPS)],
                      out.at[cid, r, pl.ds(sid * _RPS, _RPS)])
      if with_deg:
        pltpu.sync_copy(accd.at[pl.ds(sid * _RPS, _RPS)],
                        outd.at[cid, r, pl.ds(sid * _RPS, _RPS)])
      plsc.subcore_barrier()

  return functools.partial(
      pl.kernel, mesh=mesh, out_type=out_type, scratch_types=scratch,
      compiler_params=pltpu.CompilerParams(use_tc_tiling_on_sc=False))(body)


_seg128 = _seg_sum_builder(_H1, True, nbuf=2)
_seg64 = _seg_sum_builder(_H2, False, nbuf=5)


def _score_kernel():
  mesh = plsc.VectorSubcoreMesh(core_axis_name="c", subcore_axis_name="s")
  lpw = _L // _NW

  @functools.partial(
      pl.kernel, mesh=mesh,
      out_type=jax.ShapeDtypeStruct((_R * _L,), jnp.float32),
      scratch_types=[
          pltpu.VMEM((_N,), jnp.float32),
          pltpu.VMEM((_N,), jnp.float32),
          pltpu.VMEM((lpw,), jnp.int32),
          pltpu.VMEM((lpw,), jnp.int32),
          pltpu.VMEM((16,), jnp.float32),
          pltpu.VMEM((16,), jnp.float32),
          pltpu.VMEM((lpw,), jnp.float32),
      ],
      compiler_params=pltpu.CompilerParams(needs_layout_passes=False))
  def body(p0, p1, hidx, tidx, relr, reli, out,
           p0_v, p1_v, hi_v, ti_v, rr_v, ri_v, sc_v):
    cid = lax.axis_index("c")
    sid = lax.axis_index("s")
    wid = cid * _NS + sid
    pltpu.sync_copy(p0, p0_v)
    pltpu.sync_copy(p1, p1_v)
    for r in range(_R):
      pltpu.sync_copy(relr.at[pl.ds(r * 16, 16)], rr_v)
      pltpu.sync_copy(reli.at[pl.ds(r * 16, 16)], ri_v)
      pltpu.sync_copy(hidx.at[pl.ds((r * _NW + wid) * lpw, lpw)], hi_v)
      pltpu.sync_copy(tidx.at[pl.ds((r * _NW + wid) * lpw, lpw)], ti_v)
      rr = rr_v[...]
      ri = ri_v[...]
      for g in range(lpw // 16):
        hvec = hi_v[pl.ds(g * 16, 16)]
        tvec = ti_v[pl.ds(g * 16, 16)]
        h0 = plsc.load_gather(p0_v, [hvec])
        h1 = plsc.load_gather(p1_v, [hvec])
        t0 = plsc.load_gather(p0_v, [tvec])
        t1 = plsc.load_gather(p1_v, [tvec])
        s = rr * (h0 * t0 + h1 * t1) + ri * (h0 * t1 - h1 * t0)
        sc_v[pl.ds(g * 16, 16)] = s
      pltpu.sync_copy(sc_v, out.at[pl.ds((r * _NW + wid) * lpw, lpw)])

  return body


_score = _score_kernel()


def _full(shape):
  return pl.BlockSpec(shape, lambda i: (0,) * len(shape))


def _tc0_body(x_ref, w1_ref, wr1_ref, b1_ref, y1a_ref, y1b_ref, y1c_ref,
              root1_ref):
  x = x_ref[...]
  for r, yref in enumerate((y1a_ref, y1b_ref, y1c_ref)):
    yref[...] = jnp.dot(x, w1_ref[r], preferred_element_type=jnp.float32)
    root1_ref[r] = (jnp.dot(x, wr1_ref[r], preferred_element_type=jnp.float32)
                    + b1_ref[r])


def _gru_block(out_h, past, wi_ref, wh_ref, bi_ref, bh_ref, h):
  gi = jnp.dot(out_h, wi_ref[...], preferred_element_type=jnp.float32) + bi_ref[...]
  gh = jnp.dot(past, wh_ref[...], preferred_element_type=jnp.float32) + bh_ref[...]
  ir, iz, inn = gi[:, :h], gi[:, h:2 * h], gi[:, 2 * h:]
  hr, hz, hn = gh[:, :h], gh[:, h:2 * h], gh[:, 2 * h:]
  rg = jax.nn.sigmoid(ir + hr)
  zg = jax.nn.sigmoid(iz + hz)
  ng = jnp.tanh(inn + rg * hn)
  return (1.0 - zg) * ng + zg * past


def _layer_body(h, snap_ref, aggp_ref, degp_ref, root_ref, past_ref,
                wi_ref, wh_ref, bi_ref, bh_ref, aw_ref, ab_ref, aq_ref,
                cur_ref, wp_ref):
  snap0 = snap_ref[0:1, 0:1]
  lanes = lax.broadcasted_iota(jnp.int32, (8, 128), 1)
  acc = jnp.zeros((8, 128), jnp.float32)
  for r in range(_R):
    agg = aggp_ref[0, r] + aggp_ref[1, r]
    deg = degp_ref[0, r, :, 0:1] + degp_ref[1, r, :, 0:1]
    agg = agg / jnp.maximum(deg, 1.0)
    out_h = jnp.maximum(agg + root_ref[r], 0.0)
    g = _gru_block(out_h, past_ref[r], wi_ref, wh_ref, bi_ref, bh_ref, h)
    cur = jnp.where(snap0 == 0.0, out_h, g)
    cur_ref[r] = cur
    w = jnp.dot(jnp.tanh(jnp.dot(cur, aw_ref[...],
                                 preferred_element_type=jnp.float32)
                         + ab_ref[...]),
                aq_ref[...], preferred_element_type=jnp.float32)
    s = jnp.sum(w[:, 0:1])
    acc = acc + jnp.where(lanes == r, s, 0.0)

  @pl.when(pl.program_id(0) == 0)
  def _init():
    wp_ref[...] = jnp.zeros((8, 128), jnp.float32)

  wp_ref[...] += acc


def _tc2_body(cur1_ref, beta_ref, w2_ref, wr2_ref, b2_ref,
              y2a_ref, y2b_ref, y2c_ref, root2_ref):
  h1 = cur1_ref[0] * beta_ref[0:1, 0:1]
  h1 = h1 + cur1_ref[1] * beta_ref[1:2, 0:1]
  h1 = h1 + cur1_ref[2] * beta_ref[2:3, 0:1]
  for r, yref in enumerate((y2a_ref, y2b_ref, y2c_ref)):
    yref[...] = jnp.dot(h1, w2_ref[r], preferred_element_type=jnp.float32)
    root2_ref[r] = (jnp.dot(h1, wr2_ref[r], preferred_element_type=jnp.float32)
                    + b2_ref[r])


def _tc4_body(cur2_ref, beta_ref, pw_ref, pb_ref, out_ref):
  h2 = cur2_ref[0] * beta_ref[0:1, 0:1]
  h2 = h2 + cur2_ref[1] * beta_ref[1:2, 0:1]
  h2 = h2 + cur2_ref[2] * beta_ref[2:3, 0:1]
  out_ref[...] = (jnp.dot(h2, pw_ref[...], preferred_element_type=jnp.float32)
                  + pb_ref[...])


def kernel(x, edge_index, edge_label_index, snap, past1, past2, W1, Wr1, b1,
           W2, Wr2, b2, g1_Wi, g1_Wh, g1_bi, g1_bh, g2_Wi, g2_Wh, g2_bi,
           g2_bh, a1_W, a1_b, a1_q, a2_W, a2_b, a2_q, post_W, post_b,
           rel_emb):
  f32 = jnp.float32
  snapf = jnp.full((1, 128), snap, f32)
  src = edge_index[:, 0, :].reshape(_R, _NW, _NBATCH, _IB, _CW)
  dst = edge_index[:, 1, :].reshape(_R, _NW, _NBATCH, _IB, _CW)
  zw1 = jnp.zeros((_RPS, _H1), f32)
  zw2 = jnp.zeros((_RPS, _H2), f32)
  z16 = jnp.zeros((_RPS, 16), f32)
  ones = jnp.zeros((_CW, 16), f32).at[:, 0].set(1.0)

  nblk = pl.BlockSpec((_NB, _D), lambda i: (i, 0))

  y1a, y1b, y1c, root1 = pl.pallas_call(
      _tc0_body,
      grid=(_GRID,),
      in_specs=[nblk, _full((_R, _D, _H1)), _full((_R, _D, _H1)),
                _full((_R, 1, _H1))],
      out_specs=[nblk] * 3 + [pl.BlockSpec((_R, _NB, _H1), lambda i: (0, i, 0))],
      out_shape=[jax.ShapeDtypeStruct((_N, _H1), f32)] * 3
      + [jax.ShapeDtypeStruct((_R, _N, _H1), f32)],
  )(x, W1, Wr1, b1[:, None, :])

  agg1p, degp = _seg128(y1a, y1b, y1c, src, dst, zw1, z16, ones)

  layer1 = functools.partial(_layer_body, _H1)
  cur1, wp1 = pl.pallas_call(
      layer1,
      grid=(_GRID,),
      in_specs=[
          _full((1, 128)),
          pl.BlockSpec((_NC, _R, _NB, _H1), lambda i: (0, 0, i, 0)),
          pl.BlockSpec((_NC, _R, _NB, 16), lambda i: (0, 0, i, 0)),
          pl.BlockSpec((_R, _NB, _H1), lambda i: (0, i, 0)),
          pl.BlockSpec((_R, _NB, _H1), lambda i: (0, i, 0)),
          _full((_H1, 3 * _H1)), _full((_H1, 3 * _H1)),
          _full((1, 3 * _H1)), _full((1, 3 * _H1)),
          _full((_H1, _H1)), _full((1, _H1)), _full((_H1, 8)),
      ],
      out_specs=[pl.BlockSpec((_R, _NB, _H1), lambda i: (0, i, 0)),
                 pl.BlockSpec((8, 128), lambda i: (0, 0))],
      out_shape=[jax.ShapeDtypeStruct((_R, _N, _H1), f32),
                 jax.ShapeDtypeStruct((8, 128), f32)],
  )(snapf, agg1p, degp, root1, past1, g1_Wi, g1_Wh, g1_bi[None, :],
    g1_bh[None, :], a1_W, a1_b[None, :],
    jnp.zeros((_H1, 8), f32).at[:, 0].set(a1_q))

  beta1 = jax.nn.softmax(wp1[0, :_R] / _N)
  beta1b = jnp.broadcast_to(beta1[:, None], (_R, 128))

  h2blk = pl.BlockSpec((_NB, _H2), lambda i: (i, 0))
  y2a, y2b, y2c, root2 = pl.pallas_call(
      _tc2_body,
      grid=(_GRID,),
      in_specs=[pl.BlockSpec((_R, _NB, _H1), lambda i: (0, i, 0)),
                _full((_R, 128)), _full((_R, _H1, _H2)),
                _full((_R, _H1, _H2)), _full((_R, 1, _H2))],
      out_specs=[h2blk] * 3 + [pl.BlockSpec((_R, _NB, _H2), lambda i: (0, i, 0))],
      out_shape=[jax.ShapeDtypeStruct((_N, _H2), f32)] * 3
      + [jax.ShapeDtypeStruct((_R, _N, _H2), f32)],
  )(cur1, beta1b, W2, Wr2, b2[:, None, :])

  (agg2p,) = _seg64(y2a, y2b, y2c, src, dst, zw2)

  layer2 = functools.partial(_layer_body, _H2)
  cur2, wp2 = pl.pallas_call(
      layer2,
      grid=(_GRID,),
      in_specs=[
          _full((1, 128)),
          pl.BlockSpec((_NC, _R, _NB, _H2), lambda i: (0, 0, i, 0)),
          pl.BlockSpec((_NC, _R, _NB, 16), lambda i: (0, 0, i, 0)),
          pl.BlockSpec((_R, _NB, _H2), lambda i: (0, i, 0)),
          pl.BlockSpec((_R, _NB, _H2), lambda i: (0, i, 0)),
          _full((_H2, 3 * _H2)), _full((_H2, 3 * _H2)),
          _full((1, 3 * _H2)), _full((1, 3 * _H2)),
          _full((_H2, _H2)), _full((1, _H2)), _full((_H2, 8)),
      ],
      out_specs=[pl.BlockSpec((_R, _NB, _H2), lambda i: (0, i, 0)),
                 pl.BlockSpec((8, 128), lambda i: (0, 0))],
      out_shape=[jax.ShapeDtypeStruct((_R, _N, _H2), f32),
                 jax.ShapeDtypeStruct((8, 128), f32)],
  )(snapf, agg2p, degp, root2, past2, g2_Wi, g2_Wh, g2_bi[None, :],
    g2_bh[None, :], a2_W, a2_b[None, :],
    jnp.zeros((_H2, 8), f32).at[:, 0].set(a2_q))

  beta2 = jax.nn.softmax(wp2[0, :_R] / _N)
  beta2b = jnp.broadcast_to(beta2[:, None], (_R, 128))

  pwp = jnp.zeros((_H2, 16), f32).at[:, :2].set(post_W)
  pbp = jnp.zeros((1, 16), f32).at[0, :2].set(post_b)
  outp = pl.pallas_call(
      _tc4_body,
      grid=(_GRID,),
      in_specs=[pl.BlockSpec((_R, _NB, _H2), lambda i: (0, i, 0)),
                _full((_R, 128)), _full((_H2, 16)), _full((1, 16))],
      out_specs=pl.BlockSpec((_NB, 16), lambda i: (i, 0)),
      out_shape=jax.ShapeDtypeStruct((_N, 16), f32),
  )(cur2, beta2b, pwp, pbp)

  hidx = edge_label_index[:, 0, :].reshape(-1)
  tidx = edge_label_index[:, 1, :].reshape(-1)
  relr = jnp.broadcast_to(rel_emb[:, 0:1], (_R, 16)).reshape(-1)
  reli = jnp.broadcast_to(rel_emb[:, 1:2], (_R, 16)).reshape(-1)
  scores = _score(outp[:, 0], outp[:, 1], hidx, tidx, relr, reli)

  return scores.reshape(_R, _L), cur1, cur2


# speedup vs baseline: 8.5784x; 1.0026x over previous
import functools

import jax
import jax.numpy as jnp
from jax import lax
from jax.experimental import pallas as pl
from jax.experimental.pallas import tpu as pltpu
from jax.experimental.pallas import tpu_sc as plsc

_N = 10000
_D = 128
_H1 = 128
_H2 = 64
_R = 3
_E = 320000
_L = 8192

_NC = 2
_NS = 16
_NW = _NC * _NS
_EPW = _E // _NW
_CW = 125
_CH = _EPW // _CW
_IB = 10
_NBATCH = _CH // _IB
_NPAD = 10240
_RPS = _NPAD // _NS
_NB = 1000
_GRID = _N // _NB


def _seg_sum_builder(width, with_deg, nbuf=2):
  mesh = plsc.VectorSubcoreMesh(core_axis_name="c", subcore_axis_name="s")
  out_type = [jax.ShapeDtypeStruct((_NC, _R, _NPAD, width), jnp.float32)]
  scratch = [
      pltpu.VMEM_SHARED((_NPAD, width), jnp.float32),
      pltpu.VMEM((_IB, _CW), jnp.int32),
      pltpu.VMEM((_IB, _CW), jnp.int32),
  ]
  scratch += [pltpu.VMEM((_CW, width), jnp.float32)] * nbuf
  scratch += [pltpu.SemaphoreType.DMA] * nbuf
  if nbuf > 2:
    scratch += [pltpu.SemaphoreType.DMA] * nbuf
  if with_deg:
    out_type.append(jax.ShapeDtypeStruct((_NC, _R, _NPAD, 16), jnp.float32))
    scratch += [
        pltpu.VMEM_SHARED((_NPAD, 16), jnp.float32),
        pltpu.VMEM((_CW, 16), jnp.float32),
        pltpu.SemaphoreType.DMA,
    ]

  def body(*refs):
    y0, y1, y2, src, dst, zw = refs[:6]
    k = 6
    if with_deg:
      z16, ones = refs[k:k + 2]
      k += 2
    out = refs[k]
    k += 1
    if with_deg:
      outd = refs[k]
      k += 1
    acc, sidx, didx = refs[k:k + 3]
    k += 3
    rows = refs[k:k + nbuf]
    k += nbuf
    gsem = refs[k:k + nbuf]
    k += nbuf
    if nbuf > 2:
      ssem = refs[k:k + nbuf]
      k += nbuf
    if with_deg:
      accd, ones_v, dsem = refs[k:k + 3]
    cid = lax.axis_index("c")
    sid = lax.axis_index("s")
    wid = cid * _NS + sid
    if with_deg:
      pltpu.sync_copy(ones, ones_v)
    for r, yr in enumerate((y0, y1, y2)):
      pltpu.sync_copy(zw, acc.at[pl.ds(sid * _RPS, _RPS)])
      if with_deg:
        pltpu.sync_copy(z16, accd.at[pl.ds(sid * _RPS, _RPS)])
      plsc.subcore_barrier()

      if nbuf == 2:
        def batch(b, _, yr=yr, r=r):
          pltpu.sync_copy(src.at[r, wid, b], sidx)
          pltpu.sync_copy(dst.at[r, wid, b], didx)
          pltpu.async_copy(yr.at[sidx.at[0]], rows[0], gsem[0])

          def pair(p, _):
            c0 = 2 * p
            c1 = c0 + 1
            pltpu.async_copy(yr.at[sidx.at[c1]], rows[1], gsem[1])
            pltpu.make_async_copy(yr.at[sidx.at[c0]], rows[0], gsem[0]).wait()
            pltpu.sync_copy(rows[0], acc.at[didx.at[c0]], add=True)
            if with_deg:
              pltpu.async_copy(ones_v, accd.at[didx.at[c0]], dsem, add=True)

            @pl.when(c1 + 1 < _IB)
            def _prefetch():
              pltpu.async_copy(yr.at[sidx.at[c1 + 1]], rows[0], gsem[0])

            pltpu.make_async_copy(yr.at[sidx.at[c1]], rows[1], gsem[1]).wait()
            pltpu.sync_copy(rows[1], acc.at[didx.at[c1]], add=True)
            if with_deg:
              pltpu.async_copy(ones_v, accd.at[didx.at[c1]], dsem, add=True)
            return _

          res = lax.fori_loop(0, _IB // 2, pair, _)
          if with_deg:
            pltpu.make_async_copy(
                outd.at[0, 0, pl.ds(0, _IB * _CW)],
                accd.at[pl.ds(0, _IB * _CW)], dsem).wait()
          return res
      else:
        def batch(b, _, yr=yr, r=r):
          pltpu.sync_copy(src.at[r, wid, b], sidx)
          pltpu.sync_copy(dst.at[r, wid, b], didx)
          pltpu.async_copy(yr.at[sidx.at[0]], rows[0], gsem[0])
          pltpu.async_copy(yr.at[sidx.at[1]], rows[1], gsem[1])
          for c in range(_IB):
            nxt = c + 2
            if nxt < _IB:
              j = nxt % nbuf
              if nxt >= nbuf:
                pltpu.make_async_copy(
                    rows[j], acc.at[didx.at[0]], ssem[j]).wait()
              pltpu.async_copy(yr.at[sidx.at[nxt]], rows[j], gsem[j])
            i = c % nbuf
            pltpu.make_async_copy(yr.at[sidx.at[c]], rows[i], gsem[i]).wait()
            pltpu.async_copy(rows[i], acc.at[didx.at[c]], ssem[i], add=True)
          for j in range(nbuf):
            pltpu.make_async_copy(rows[j], acc.at[didx.at[0]], ssem[j]).wait()
          return _

      lax.fori_loop(0, _NBATCH, batch, None)
      plsc.subcore_barrier()
      pltpu.sync_copy(acc.at[pl.ds(sid * _RPS, _RPS)],
                      out.at[cid, r, pl.ds(sid * _RPS, _RPS)])
      if with_deg:
        pltpu.sync_copy(accd.at[pl.ds(sid * _RPS, _RPS)],
                        outd.at[cid, r, pl.ds(sid * _RPS, _RPS)])
      plsc.subcore_barrier()

  return functools.partial(
      pl.kernel, mesh=mesh, out_type=out_type, scratch_types=scratch,
      compiler_params=pltpu.CompilerParams(use_tc_tiling_on_sc=False))(body)


_seg128 = _seg_sum_builder(_H1, True, nbuf=2)
_seg64 = _seg_sum_builder(_H2, False, nbuf=5)


def _score_kernel():
  mesh = plsc.VectorSubcoreMesh(core_axis_name="c", subcore_axis_name="s")
  lpw = _L // _NW

  @functools.partial(
      pl.kernel, mesh=mesh,
      out_type=jax.ShapeDtypeStruct((_R * _L,), jnp.float32),
      scratch_types=[
          pltpu.VMEM((_N,), jnp.float32),
          pltpu.VMEM((_N,), jnp.float32),
          pltpu.VMEM((lpw,), jnp.int32),
          pltpu.VMEM((lpw,), jnp.int32),
          pltpu.VMEM((16,), jnp.float32),
          pltpu.VMEM((16,), jnp.float32),
          pltpu.VMEM((lpw,), jnp.float32),
      ],
      compiler_params=pltpu.CompilerParams(needs_layout_passes=False))
  def body(p0, p1, hidx, tidx, relr, reli, out,
           p0_v, p1_v, hi_v, ti_v, rr_v, ri_v, sc_v):
    cid = lax.axis_index("c")
    sid = lax.axis_index("s")
    wid = cid * _NS + sid
    pltpu.sync_copy(p0, p0_v)
    pltpu.sync_copy(p1, p1_v)
    for r in range(_R):
      pltpu.sync_copy(relr.at[pl.ds(r * 16, 16)], rr_v)
      pltpu.sync_copy(reli.at[pl.ds(r * 16, 16)], ri_v)
      pltpu.sync_copy(hidx.at[pl.ds((r * _NW + wid) * lpw, lpw)], hi_v)
      pltpu.sync_copy(tidx.at[pl.ds((r * _NW + wid) * lpw, lpw)], ti_v)
      rr = rr_v[...]
      ri = ri_v[...]
      for g in range(lpw // 16):
        hvec = hi_v[pl.ds(g * 16, 16)]
        tvec = ti_v[pl.ds(g * 16, 16)]
        h0 = plsc.load_gather(p0_v, [hvec])
        h1 = plsc.load_gather(p1_v, [hvec])
        t0 = plsc.load_gather(p0_v, [tvec])
        t1 = plsc.load_gather(p1_v, [tvec])
        s = rr * (h0 * t0 + h1 * t1) + ri * (h0 * t1 - h1 * t0)
        sc_v[pl.ds(g * 16, 16)] = s
      pltpu.sync_copy(sc_v, out.at[pl.ds((r * _NW + wid) * lpw, lpw)])

  return body


_score = _score_kernel()


def _full(shape):
  return pl.BlockSpec(shape, lambda i: (0,) * len(shape))


def _tc0_body(x_ref, w1_ref, wr1_ref, b1_ref, y1a_ref, y1b_ref, y1c_ref,
              root1_ref):
  x = x_ref[...]
  for r, yref in enumerate((y1a_ref, y1b_ref, y1c_ref)):
    yref[...] = jnp.dot(x, w1_ref[r], preferred_element_type=jnp.float32)
    root1_ref[r] = (jnp.dot(x, wr1_ref[r], preferred_element_type=jnp.float32)
                    + b1_ref[r])


def _gru_block(out_h, past, wi_ref, wh_ref, bi_ref, bh_ref, h):
  gi = jnp.dot(out_h, wi_ref[...], preferred_element_type=jnp.float32) + bi_ref[...]
  gh = jnp.dot(past, wh_ref[...], preferred_element_type=jnp.float32) + bh_ref[...]
  ir, iz, inn = gi[:, :h], gi[:, h:2 * h], gi[:, 2 * h:]
  hr, hz, hn = gh[:, :h], gh[:, h:2 * h], gh[:, 2 * h:]
  rg = jax.nn.sigmoid(ir + hr)
  zg = jax.nn.sigmoid(iz + hz)
  ng = jnp.tanh(inn + rg * hn)
  return (1.0 - zg) * ng + zg * past


def _layer_body(h, snap_ref, aggp_ref, degp_ref, root_ref, past_ref,
                wi_ref, wh_ref, bi_ref, bh_ref, aw_ref, ab_ref, aq_ref,
                cur_ref, wp_ref):
  snap0 = snap_ref[0:1, 0:1]
  lanes = lax.broadcasted_iota(jnp.int32, (8, 128), 1)
  acc = jnp.zeros((8, 128), jnp.float32)
  for r in range(_R):
    agg = aggp_ref[0, r] + aggp_ref[1, r]
    deg = degp_ref[0, r, :, 0:1] + degp_ref[1, r, :, 0:1]
    agg = agg / jnp.maximum(deg, 1.0)
    out_h = jnp.maximum(agg + root_ref[r], 0.0)
    g = _gru_block(out_h, past_ref[r], wi_ref, wh_ref, bi_ref, bh_ref, h)
    cur = jnp.where(snap0 == 0.0, out_h, g)
    cur_ref[r] = cur
    w = jnp.dot(jnp.tanh(jnp.dot(cur, aw_ref[...],
                                 preferred_element_type=jnp.float32)
                         + ab_ref[...]),
                aq_ref[...], preferred_element_type=jnp.float32)
    s = jnp.sum(w[:, 0:1])
    acc = acc + jnp.where(lanes == r, s, 0.0)

  @pl.when(pl.program_id(0) == 0)
  def _init():
    wp_ref[...] = jnp.zeros((8, 128), jnp.float32)

  wp_ref[...] += acc


def _tc2_body(cur1_ref, beta_ref, w2_ref, wr2_ref, b2_ref,
              y2a_ref, y2b_ref, y2c_ref, root2_ref):
  h1 = cur1_ref[0] * beta_ref[0:1, 0:1]
  h1 = h1 + cur1_ref[1] * beta_ref[1:2, 0:1]
  h1 = h1 + cur1_ref[2] * beta_ref[2:3, 0:1]
  for r, yref in enumerate((y2a_ref, y2b_ref, y2c_ref)):
    yref[...] = jnp.dot(h1, w2_ref[r], preferred_element_type=jnp.float32)
    root2_ref[r] = (jnp.dot(h1, wr2_ref[r], preferred_element_type=jnp.float32)
                    + b2_ref[r])


def _tc4_body(cur2_ref, beta_ref, pw_ref, pb_ref, out_ref):
  h2 = cur2_ref[0] * beta_ref[0:1, 0:1]
  h2 = h2 + cur2_ref[1] * beta_ref[1:2, 0:1]
  h2 = h2 + cur2_ref[2] * beta_ref[2:3, 0:1]
  out_ref[...] = (jnp.dot(h2, pw_ref[...], preferred_element_type=jnp.float32)
                  + pb_ref[...])


def kernel(x, edge_index, edge_label_index, snap, past1, past2, W1, Wr1, b1,
           W2, Wr2, b2, g1_Wi, g1_Wh, g1_bi, g1_bh, g2_Wi, g2_Wh, g2_bi,
           g2_bh, a1_W, a1_b, a1_q, a2_W, a2_b, a2_q, post_W, post_b,
           rel_emb):
  f32 = jnp.float32
  snapf = jnp.full((1, 128), snap, f32)
  src = edge_index[:, 0, :].reshape(_R, _NW, _NBATCH, _IB, _CW)
  dst = edge_index[:, 1, :].reshape(_R, _NW, _NBATCH, _IB, _CW)
  zw1 = jnp.zeros((_RPS, _H1), f32)
  zw2 = jnp.zeros((_RPS, _H2), f32)
  z16 = jnp.zeros((_RPS, 16), f32)
  ones = jnp.zeros((_CW, 16), f32).at[:, 0].set(1.0)

  nblk = pl.BlockSpec((_NB, _D), lambda i: (i, 0))

  y1a, y1b, y1c, root1 = pl.pallas_call(
      _tc0_body,
      grid=(_GRID,),
      in_specs=[nblk, _full((_R, _D, _H1)), _full((_R, _D, _H1)),
                _full((_R, 1, _H1))],
      out_specs=[nblk] * 3 + [pl.BlockSpec((_R, _NB, _H1), lambda i: (0, i, 0))],
      out_shape=[jax.ShapeDtypeStruct((_N, _H1), f32)] * 3
      + [jax.ShapeDtypeStruct((_R, _N, _H1), f32)],
  )(x, W1, Wr1, b1[:, None, :])

  agg1p, degp = _seg128(y1a, y1b, y1c, src, dst, zw1, z16, ones)

  layer1 = functools.partial(_layer_body, _H1)
  cur1, wp1 = pl.pallas_call(
      layer1,
      grid=(_GRID,),
      in_specs=[
          _full((1, 128)),
          pl.BlockSpec((_NC, _R, _NB, _H1), lambda i: (0, 0, i, 0)),
          pl.BlockSpec((_NC, _R, _NB, 16), lambda i: (0, 0, i, 0)),
          pl.BlockSpec((_R, _NB, _H1), lambda i: (0, i, 0)),
          pl.BlockSpec((_R, _NB, _H1), lambda i: (0, i, 0)),
          _full((_H1, 3 * _H1)), _full((_H1, 3 * _H1)),
          _full((1, 3 * _H1)), _full((1, 3 * _H1)),
          _full((_H1, _H1)), _full((1, _H1)), _full((_H1, 8)),
      ],
      out_specs=[pl.BlockSpec((_R, _NB, _H1), lambda i: (0, i, 0)),
                 pl.BlockSpec((8, 128), lambda i: (0, 0))],
      out_shape=[jax.ShapeDtypeStruct((_R, _N, _H1), f32),
                 jax.ShapeDtypeStruct((8, 128), f32)],
  )(snapf, agg1p, degp, root1, past1, g1_Wi, g1_Wh, g1_bi[None, :],
    g1_bh[None, :], a1_W, a1_b[None, :],
    jnp.zeros((_H1, 8), f32).at[:, 0].set(a1_q))

  beta1 = jax.nn.softmax(wp1[0, :_R] / _N)
  beta1b = jnp.broadcast_to(beta1[:, None], (_R, 128))

  h2blk = pl.BlockSpec((_NB, _H2), lambda i: (i, 0))
  y2a, y2b, y2c, root2 = pl.pallas_call(
      _tc2_body,
      grid=(_GRID,),
      in_specs=[pl.BlockSpec((_R, _NB, _H1), lambda i: (0, i, 0)),
                _full((_R, 128)), _full((_R, _H1, _H2)),
                _full((_R, _H1, _H2)), _full((_R, 1, _H2))],
      out_specs=[h2blk] * 3 + [pl.BlockSpec((_R, _NB, _H2), lambda i: (0, i, 0))],
      out_shape=[jax.ShapeDtypeStruct((_N, _H2), f32)] * 3
      + [jax.ShapeDtypeStruct((_R, _N, _H2), f32)],
  )(cur1, beta1b, W2, Wr2, b2[:, None, :])

  (agg2p,) = _seg64(y2a, y2b, y2c, src, dst, zw2)

  layer2 = functools.partial(_layer_body, _H2)
  cur2, wp2 = pl.pallas_call(
      layer2,
      grid=(_GRID,),
      in_specs=[
          _full((1, 128)),
          pl.BlockSpec((_NC, _R, _NB, _H2), lambda i: (0, 0, i, 0)),
          pl.BlockSpec((_NC, _R, _NB, 16), lambda i: (0, 0, i, 0)),
          pl.BlockSpec((_R, _NB, _H2), lambda i: (0, i, 0)),
          pl.BlockSpec((_R, _NB, _H2), lambda i: (0, i, 0)),
          _full((_H2, 3 * _H2)), _full((_H2, 3 * _H2)),
          _full((1, 3 * _H2)), _full((1, 3 * _H2)),
          _full((_H2, _H2)), _full((1, _H2)), _full((_H2, 8)),
      ],
      out_specs=[pl.BlockSpec((_R, _NB, _H2), lambda i: (0, i, 0)),
                 pl.BlockSpec((8, 128), lambda i: (0, 0))],
      out_shape=[jax.ShapeDtypeStruct((_R, _N, _H2), f32),
                 jax.ShapeDtypeStruct((8, 128), f32)],
  )(snapf, agg2p, degp, root2, past2, g2_Wi, g2_Wh, g2_bi[None, :],
    g2_bh[None, :], a2_W, a2_b[None, :],
    jnp.zeros((_H2, 8), f32).at[:, 0].set(a2_q))

  beta2 = jax.nn.softmax(wp2[0, :_R] / _N)
  beta2b = jnp.broadcast_to(beta2[:, None], (_R, 128))

  pwp = jnp.zeros((_H2, 16), f32).at[:, :2].set(post_W)
  pbp = jnp.zeros((1, 16), f32).at[0, :2].set(post_b)
  outp = pl.pallas_call(
      _tc4_body,
      grid=(_GRID,),
      in_specs=[pl.BlockSpec((_R, _NB, _H2), lambda i: (0, i, 0)),
                _full((_R, 128)), _full((_H2, 16)), _full((1, 16))],
      out_specs=pl.BlockSpec((_NB, 16), lambda i: (i, 0)),
      out_shape=jax.ShapeDtypeStruct((_N, 16), f32),
  )(cur2, beta2b, pwp, pbp)

  hidx = edge_label_index[:, 0, :].reshape(-1)
  tidx = edge_label_index[:, 1, :].reshape(-1)
  relr = jnp.broadcast_to(rel_emb[:, 0:1], (_R, 16)).reshape(-1)
  reli = jnp.broadcast_to(rel_emb[:, 1:2], (_R, 16)).reshape(-1)
  scores = _score(outp[:, 0], outp[:, 1], hidx, tidx, relr, reli)

  return scores.reshape(_R, _L), cur1, cur2
